# baseline (speedup 1.0000x reference)
import functools

import jax
import jax.numpy as jnp
from jax import lax
from jax.experimental import pallas as pl
from jax.experimental.pallas import tpu as pltpu
from jax.experimental.pallas import tpu_sc as plsc

N_NODES = 10000
N_PAD = 10240
E = 320000
D_IN = 128
D_TAB = 144
NCLS = 47

NC = 2
NS = 16
NW = NC * NS
B = 128
NCHUNK = 80
EPW = NCHUNK * B
E_PAD = NW * EPW
ROWS_PT = N_PAD // NS


def _make_agg(D):
    mesh = plsc.VectorSubcoreMesh(core_axis_name="c", subcore_axis_name="s")

    @functools.partial(
        pl.kernel,
        mesh=mesh,
        compiler_params=pltpu.CompilerParams(use_tc_tiling_on_sc=False),
        out_type=jax.ShapeDtypeStruct((NC, N_PAD, D), jnp.float32),
        scratch_types=[
            pltpu.VMEM((2, B), jnp.int32),
            pltpu.VMEM((2, B), jnp.int32),
            pltpu.VMEM((B, D), jnp.float32),
            pltpu.VMEM((B, D), jnp.float32),
            pltpu.VMEM_SHARED((N_PAD, D), jnp.float32),
            pltpu.SemaphoreType.DMA,
            pltpu.SemaphoreType.DMA,
            pltpu.SemaphoreType.DMA,
            pltpu.SemaphoreType.DMA,
        ],
    )
    def agg(table_hbm, sd_hbm, out_hbm,
            sd0, sd1, buf0, buf1, acc_sh, isem0, isem1, gsem0, gsem1):
        c = lax.axis_index("c")
        s = lax.axis_index("s")
        wid = s * NC + c

        def zrow(i, carry):
            for b in range(D // 16):
                buf0[i, pl.ds(b * 16, 16)] = jnp.zeros((16,), jnp.float32)
            return carry

        lax.fori_loop(0, B, zrow, 0)
        for j in range(ROWS_PT // B):
            pltpu.sync_copy(buf0, acc_sh.at[pl.ds(s * ROWS_PT + j * B, B)])
        plsc.subcore_barrier()

        row0 = wid * NCHUNK

        def idx_load(k, sd, isem):
            pltpu.async_copy(sd_hbm.at[row0 + k], sd, isem)

        def idx_wait(sd, isem):
            pltpu.make_async_copy(sd_hbm.at[row0], sd, isem).wait()

        def gather(sd, buf, gsem):
            pltpu.async_copy(table_hbm.at[sd.at[0]], buf, gsem)

        def gwait(sd, buf, gsem):
            pltpu.make_async_copy(table_hbm.at[sd.at[0]], buf, gsem).wait()

        def body(k, carry):
            idx_load(k, sd0, isem0)
            idx_wait(sd0, isem0)
            gather(sd0, buf0, gsem0)
            gwait(sd0, buf0, gsem0)
            pltpu.sync_copy(buf0, acc_sh.at[sd0.at[1]], add=True)
            return carry

        lax.fori_loop(0, NCHUNK, body, 0)
        plsc.subcore_barrier()
        pltpu.sync_copy(
            acc_sh.at[pl.ds(s * ROWS_PT, ROWS_PT)],
            out_hbm.at[c, pl.ds(s * ROWS_PT, ROWS_PT)],
        )

    return agg


_agg_tab = _make_agg(D_TAB)
_agg_feat = _make_agg(D_IN)


def _combine_body(p_ref, m_ref, d_ref):
    s = p_ref[0] + p_ref[1]
    deg = s[:, 128:129]
    m_ref[...] = s[:, :128] / jnp.maximum(deg, 1.0)
    d_ref[...] = deg


_R1 = 2048


def _combine(p1):
    return pl.pallas_call(
        _combine_body,
        grid=(N_PAD // _R1,),
        in_specs=[pl.BlockSpec((NC, _R1, D_TAB), lambda i: (0, i, 0))],
        out_specs=[
            pl.BlockSpec((_R1, D_IN), lambda i: (i, 0)),
            pl.BlockSpec((_R1, 1), lambda i: (i, 0)),
        ],
        out_shape=[
            jax.ShapeDtypeStruct((N_PAD, D_IN), jnp.float32),
            jax.ShapeDtypeStruct((N_PAD, 1), jnp.float32),
        ],
    )(p1)


def _wcomb_body(ws1, wn1, ws2, wn2, b1, b2, wa, wb, wc, cm):
    f32 = jnp.float32
    wa[...] = jnp.dot(ws1[...], ws2[...], preferred_element_type=f32)
    wb[...] = jnp.dot(wn1[...], ws2[...], preferred_element_type=f32) + jnp.dot(
        ws1[...], wn2[...], preferred_element_type=f32
    )
    wc[...] = jnp.dot(wn1[...], wn2[...], preferred_element_type=f32)
    cm[0:1, :] = jnp.dot(b1[...], ws2[...], preferred_element_type=f32) + b2[...]
    cm[1:2, :] = jnp.dot(b1[...], wn2[...], preferred_element_type=f32)


def _wcomb(Ws1, Wn1, Ws2, Wn2, b1, b2):
    sh = jax.ShapeDtypeStruct
    return pl.pallas_call(
        _wcomb_body,
        out_shape=[
            sh((D_IN, NCLS), jnp.float32),
            sh((D_IN, NCLS), jnp.float32),
            sh((D_IN, NCLS), jnp.float32),
            sh((2, NCLS), jnp.float32),
        ],
    )(Ws1, Wn1, Ws2, Wn2, b1, b2)


_R2 = 2000


def _final_body(f_ref, m_ref, p2_ref, d_ref, wa_ref, wb_ref, wc_ref, cm_ref, o_ref):
    deg = d_ref[...]
    dmax = jnp.maximum(deg, 1.0)
    m1m = (p2_ref[0] + p2_ref[1]) / dmax
    r = (deg > 0.0).astype(jnp.float32)
    f32 = jnp.float32
    acc = jnp.dot(f_ref[...], wa_ref[...], preferred_element_type=f32)
    acc += jnp.dot(m_ref[...], wb_ref[...], preferred_element_type=f32)
    acc += jnp.dot(m1m, wc_ref[...], preferred_element_type=f32)
    acc += cm_ref[0:1, :] + r * cm_ref[1:2, :]
    o_ref[...] = acc


def _final(features, m1tab, p2, deg, wa, wb, wc, cm):
    return pl.pallas_call(
        _final_body,
        grid=(N_NODES // _R2,),
        in_specs=[
            pl.BlockSpec((_R2, D_IN), lambda i: (i, 0)),
            pl.BlockSpec((_R2, D_IN), lambda i: (i, 0)),
            pl.BlockSpec((NC, _R2, D_IN), lambda i: (0, i, 0)),
            pl.BlockSpec((_R2, 1), lambda i: (i, 0)),
            pl.BlockSpec((D_IN, NCLS), lambda i: (0, 0)),
            pl.BlockSpec((D_IN, NCLS), lambda i: (0, 0)),
            pl.BlockSpec((D_IN, NCLS), lambda i: (0, 0)),
            pl.BlockSpec((2, NCLS), lambda i: (0, 0)),
        ],
        out_specs=pl.BlockSpec((_R2, NCLS), lambda i: (i, 0)),
        out_shape=jax.ShapeDtypeStruct((N_NODES, NCLS), jnp.float32),
    )(features, m1tab, p2, deg, wa, wb, wc, cm)


def kernel(features, edge_index, W_self1, W_neigh1, b1, W_self2, W_neigh2, b2):
    src = jnp.full((E_PAD,), N_NODES, jnp.int32).at[:E].set(
        edge_index[0].astype(jnp.int32)).reshape(NW * NCHUNK, 1, B)
    dst = jnp.full((E_PAD,), N_NODES, jnp.int32).at[:E].set(
        edge_index[1].astype(jnp.int32)).reshape(NW * NCHUNK, 1, B)
    sd = jnp.concatenate([src, dst], axis=1)

    ones = jnp.ones((N_NODES, 1), jnp.float32)
    zpad = jnp.zeros((N_NODES, D_TAB - D_IN - 1), jnp.float32)
    ftab = jnp.concatenate([features, ones, zpad], axis=1)
    ftab = jnp.pad(ftab, ((0, N_PAD - N_NODES), (0, 0)))

    p1 = _agg_tab(ftab, sd)
    m1tab, deg = _combine(p1)
    p2 = _agg_feat(m1tab, sd)
    wa, wb, wc, cm = _wcomb(
        W_self1, W_neigh1, W_self2, W_neigh2,
        b1.reshape(1, -1), b2.reshape(1, -1),
    )
    return _final(features, m1tab, p2, deg, wa, wb, wc, cm)


# speedup vs baseline: 2.7674x; 2.7674x over previous
import functools

import jax
import jax.numpy as jnp
from jax import lax
from jax.experimental import pallas as pl
from jax.experimental.pallas import tpu as pltpu
from jax.experimental.pallas import tpu_sc as plsc

N_NODES = 10000
N_PAD = 10240
E = 320000
D_IN = 128
D_TAB = 144
NCLS = 47

NC = 2
NS = 16
NW = NC * NS
EPW = E // NW
B = 80
NCHUNK = EPW // B
ROWS_PT = N_PAD // NS


def _make_agg(D):
    mesh = plsc.VectorSubcoreMesh(core_axis_name="c", subcore_axis_name="s")

    @functools.partial(
        pl.kernel,
        mesh=mesh,
        compiler_params=pltpu.CompilerParams(use_tc_tiling_on_sc=False),
        out_type=jax.ShapeDtypeStruct((NC, N_PAD, D), jnp.float32),
        scratch_types=[
            pltpu.VMEM((B,), jnp.int32),
            pltpu.VMEM((B,), jnp.int32),
            pltpu.VMEM((B,), jnp.int32),
            pltpu.VMEM((B,), jnp.int32),
            pltpu.VMEM((B, D), jnp.float32),
            pltpu.VMEM((B, D), jnp.float32),
            pltpu.VMEM_SHARED((N_PAD, D), jnp.float32),
            pltpu.SemaphoreType.DMA,
            pltpu.SemaphoreType.DMA,
        ],
    )
    def agg(table_hbm, src_hbm, dst_hbm, out_hbm,
            src0, dst0, src1, dst1, buf0, buf1, acc_sh, gsem0, gsem1):
        c = lax.axis_index("c")
        s = lax.axis_index("s")
        wid = s * NC + c

        def zrow(i, carry):
            for b in range(D // 16):
                buf0[i, pl.ds(b * 16, 16)] = jnp.zeros((16,), jnp.float32)
            return carry

        lax.fori_loop(0, B, zrow, 0)
        for j in range(ROWS_PT // B):
            pltpu.sync_copy(buf0, acc_sh.at[pl.ds(s * ROWS_PT + j * B, B)])
        plsc.subcore_barrier()

        base0 = wid * EPW

        def idx_load(k, src_v, dst_v):
            pltpu.sync_copy(src_hbm.at[pl.ds(base0 + k * B, B)], src_v)
            pltpu.sync_copy(dst_hbm.at[pl.ds(base0 + k * B, B)], dst_v)

        def gather(src_v, buf, gsem):
            pltpu.async_copy(table_hbm.at[src_v], buf, gsem)

        def gwait(src_v, buf, gsem):
            pltpu.make_async_copy(table_hbm.at[src_v], buf, gsem).wait()

        def scat(dst_v, buf):
            pltpu.sync_copy(buf, acc_sh.at[dst_v], add=True)

        idx_load(0, src0, dst0)
        gather(src0, buf0, gsem0)

        def body(j, carry):
            a = 2 * j
            idx_load(a + 1, src1, dst1)
            gather(src1, buf1, gsem1)
            gwait(src0, buf0, gsem0)
            scat(dst0, buf0)
            idx_load(a + 2, src0, dst0)
            gather(src0, buf0, gsem0)
            gwait(src1, buf1, gsem1)
            scat(dst1, buf1)
            return carry

        lax.fori_loop(0, (NCHUNK - 1) // 2, body, 0)
        gwait(src0, buf0, gsem0)
        scat(dst0, buf0)

        plsc.subcore_barrier()
        pltpu.sync_copy(
            acc_sh.at[pl.ds(s * ROWS_PT, ROWS_PT)],
            out_hbm.at[c, pl.ds(s * ROWS_PT, ROWS_PT)],
        )

    return agg


_agg_tab = _make_agg(D_TAB)
_agg_feat = _make_agg(D_IN)


def _combine_body(p_ref, m_ref, d_ref):
    s = p_ref[0] + p_ref[1]
    deg = s[:, 128:129]
    m_ref[...] = s[:, :128] / jnp.maximum(deg, 1.0)
    d_ref[...] = deg


_R1 = 2048


def _combine(p1):
    return pl.pallas_call(
        _combine_body,
        grid=(N_PAD // _R1,),
        in_specs=[pl.BlockSpec((NC, _R1, D_TAB), lambda i: (0, i, 0))],
        out_specs=[
            pl.BlockSpec((_R1, D_IN), lambda i: (i, 0)),
            pl.BlockSpec((_R1, 1), lambda i: (i, 0)),
        ],
        out_shape=[
            jax.ShapeDtypeStruct((N_PAD, D_IN), jnp.float32),
            jax.ShapeDtypeStruct((N_PAD, 1), jnp.float32),
        ],
    )(p1)


def _wcomb_body(ws1, wn1, ws2, wn2, b1, b2, wa, wb, wc, cm):
    f32 = jnp.float32
    wa[...] = jnp.dot(ws1[...], ws2[...], preferred_element_type=f32)
    wb[...] = jnp.dot(wn1[...], ws2[...], preferred_element_type=f32) + jnp.dot(
        ws1[...], wn2[...], preferred_element_type=f32
    )
    wc[...] = jnp.dot(wn1[...], wn2[...], preferred_element_type=f32)
    cm[0:1, :] = jnp.dot(b1[...], ws2[...], preferred_element_type=f32) + b2[...]
    cm[1:2, :] = jnp.dot(b1[...], wn2[...], preferred_element_type=f32)


def _wcomb(Ws1, Wn1, Ws2, Wn2, b1, b2):
    sh = jax.ShapeDtypeStruct
    return pl.pallas_call(
        _wcomb_body,
        out_shape=[
            sh((D_IN, NCLS), jnp.float32),
            sh((D_IN, NCLS), jnp.float32),
            sh((D_IN, NCLS), jnp.float32),
            sh((2, NCLS), jnp.float32),
        ],
    )(Ws1, Wn1, Ws2, Wn2, b1, b2)


_R2 = 2000


def _final_body(f_ref, m_ref, p2_ref, d_ref, wa_ref, wb_ref, wc_ref, cm_ref, o_ref):
    deg = d_ref[...]
    dmax = jnp.maximum(deg, 1.0)
    m1m = (p2_ref[0] + p2_ref[1]) / dmax
    r = (deg > 0.0).astype(jnp.float32)
    f32 = jnp.float32
    acc = jnp.dot(f_ref[...], wa_ref[...], preferred_element_type=f32)
    acc += jnp.dot(m_ref[...], wb_ref[...], preferred_element_type=f32)
    acc += jnp.dot(m1m, wc_ref[...], preferred_element_type=f32)
    acc += cm_ref[0:1, :] + r * cm_ref[1:2, :]
    o_ref[...] = acc


def _final(features, m1tab, p2, deg, wa, wb, wc, cm):
    return pl.pallas_call(
        _final_body,
        grid=(N_NODES // _R2,),
        in_specs=[
            pl.BlockSpec((_R2, D_IN), lambda i: (i, 0)),
            pl.BlockSpec((_R2, D_IN), lambda i: (i, 0)),
            pl.BlockSpec((NC, _R2, D_IN), lambda i: (0, i, 0)),
            pl.BlockSpec((_R2, 1), lambda i: (i, 0)),
            pl.BlockSpec((D_IN, NCLS), lambda i: (0, 0)),
            pl.BlockSpec((D_IN, NCLS), lambda i: (0, 0)),
            pl.BlockSpec((D_IN, NCLS), lambda i: (0, 0)),
            pl.BlockSpec((2, NCLS), lambda i: (0, 0)),
        ],
        out_specs=pl.BlockSpec((_R2, NCLS), lambda i: (i, 0)),
        out_shape=jax.ShapeDtypeStruct((N_NODES, NCLS), jnp.float32),
    )(features, m1tab, p2, deg, wa, wb, wc, cm)


def kernel(features, edge_index, W_self1, W_neigh1, b1, W_self2, W_neigh2, b2):
    src = edge_index[0].astype(jnp.int32)
    dst = edge_index[1].astype(jnp.int32)

    ones = jnp.ones((N_NODES, 1), jnp.float32)
    zpad = jnp.zeros((N_NODES, D_TAB - D_IN - 1), jnp.float32)
    ftab = jnp.concatenate([features, ones, zpad], axis=1)
    ftab = jnp.pad(ftab, ((0, N_PAD - N_NODES), (0, 0)))

    p1 = _agg_tab(ftab, src, dst)
    m1tab, deg = _combine(p1)
    p2 = _agg_feat(m1tab, src, dst)
    wa, wb, wc, cm = _wcomb(
        W_self1, W_neigh1, W_self2, W_neigh2,
        b1.reshape(1, -1), b2.reshape(1, -1),
    )
    return _final(features, m1tab, p2, deg, wa, wb, wc, cm)


# speedup vs baseline: 3.2881x; 1.1882x over previous
import functools

import jax
import jax.numpy as jnp
from jax import lax
from jax.experimental import pallas as pl
from jax.experimental.pallas import tpu as pltpu
from jax.experimental.pallas import tpu_sc as plsc

N_NODES = 10000
N_PAD = 10240
E = 320000
D_IN = 128
D_TAB = 144
NCLS = 47

NC = 2
NS = 16
NW = NC * NS
EPW = E // NW
B = 80
NCHUNK = EPW // B
ROWS_PT = N_PAD // NS


def _make_agg(D):
    mesh = plsc.VectorSubcoreMesh(core_axis_name="c", subcore_axis_name="s")

    DEPTH = 3

    @functools.partial(
        pl.kernel,
        mesh=mesh,
        compiler_params=pltpu.CompilerParams(use_tc_tiling_on_sc=False),
        out_type=jax.ShapeDtypeStruct((NC, N_PAD, D), jnp.float32),
        scratch_types=[
            pltpu.VMEM((B,), jnp.int32),
            pltpu.VMEM((B,), jnp.int32),
            pltpu.VMEM((B,), jnp.int32),
            pltpu.VMEM((B,), jnp.int32),
            pltpu.VMEM((B,), jnp.int32),
            pltpu.VMEM((B,), jnp.int32),
            pltpu.VMEM((B, D), jnp.float32),
            pltpu.VMEM((B, D), jnp.float32),
            pltpu.VMEM((B, D), jnp.float32),
            pltpu.VMEM_SHARED((N_PAD, D), jnp.float32),
            pltpu.SemaphoreType.DMA,
            pltpu.SemaphoreType.DMA,
            pltpu.SemaphoreType.DMA,
            pltpu.SemaphoreType.DMA,
            pltpu.SemaphoreType.DMA,
            pltpu.SemaphoreType.DMA,
        ],
    )
    def agg(table_hbm, src_hbm, dst_hbm, out_hbm,
            sv0, sv1, sv2, dv0, dv1, dv2, buf0, buf1, buf2, acc_sh,
            g0, g1, g2, s0, s1, s2):
        c = lax.axis_index("c")
        s = lax.axis_index("s")
        wid = s * NC + c
        src_v = [sv0, sv1, sv2]
        dst_v = [dv0, dv1, dv2]
        bufs = [buf0, buf1, buf2]
        gsem = [g0, g1, g2]
        ssem = [s0, s1, s2]

        def zrow(i, carry):
            for b in range(D // 16):
                buf0[i, pl.ds(b * 16, 16)] = jnp.zeros((16,), jnp.float32)
            return carry

        lax.fori_loop(0, B, zrow, 0)
        for j in range(ROWS_PT // B):
            pltpu.sync_copy(buf0, acc_sh.at[pl.ds(s * ROWS_PT + j * B, B)])
        plsc.subcore_barrier()

        base0 = wid * EPW

        def idx_load(k, u):
            pltpu.sync_copy(src_hbm.at[pl.ds(base0 + k * B, B)], src_v[u])
            pltpu.sync_copy(dst_hbm.at[pl.ds(base0 + k * B, B)], dst_v[u])

        def gather(u):
            pltpu.async_copy(table_hbm.at[src_v[u]], bufs[u], gsem[u])

        def gwait(u):
            pltpu.make_async_copy(
                table_hbm.at[src_v[u]], bufs[u], gsem[u]).wait()

        def scat_start(u):
            pltpu.async_copy(bufs[u], acc_sh.at[dst_v[u]], ssem[u], add=True)

        def swait(u):
            pltpu.make_async_copy(
                bufs[u], acc_sh.at[dst_v[u]], ssem[u]).wait()

        for k in range(DEPTH - 1):
            idx_load(k, k)
            gather(k)

        def body(j, carry):
            for t in range(DEPTH):
                cur = DEPTH * j + t
                nxt = cur + DEPTH - 1
                u_nxt = (t + DEPTH - 1) % DEPTH

                @pl.when(cur < NCHUNK)
                def _():
                    gwait(t)
                    scat_start(t)

                @pl.when(nxt < NCHUNK)
                def _():
                    @pl.when(nxt >= DEPTH)
                    def _():
                        swait(u_nxt)

                    idx_load(nxt, u_nxt)
                    gather(u_nxt)

            return carry

        nbody = (NCHUNK + DEPTH - 1) // DEPTH
        lax.fori_loop(0, nbody, body, 0)
        for u in range(DEPTH):
            swait(u)

        plsc.subcore_barrier()
        pltpu.sync_copy(
            acc_sh.at[pl.ds(s * ROWS_PT, ROWS_PT)],
            out_hbm.at[c, pl.ds(s * ROWS_PT, ROWS_PT)],
        )

    return agg


_agg_tab = _make_agg(D_TAB)
_agg_feat = _make_agg(D_IN)


def _combine_body(p_ref, m_ref, d_ref):
    s = p_ref[0] + p_ref[1]
    deg = s[:, 128:129]
    m_ref[...] = s[:, :128] / jnp.maximum(deg, 1.0)
    d_ref[...] = deg


_R1 = 2048


def _combine(p1):
    return pl.pallas_call(
        _combine_body,
        grid=(N_PAD // _R1,),
        in_specs=[pl.BlockSpec((NC, _R1, D_TAB), lambda i: (0, i, 0))],
        out_specs=[
            pl.BlockSpec((_R1, D_IN), lambda i: (i, 0)),
            pl.BlockSpec((_R1, 1), lambda i: (i, 0)),
        ],
        out_shape=[
            jax.ShapeDtypeStruct((N_PAD, D_IN), jnp.float32),
            jax.ShapeDtypeStruct((N_PAD, 1), jnp.float32),
        ],
    )(p1)


def _wcomb_body(ws1, wn1, ws2, wn2, b1, b2, wa, wb, wc, cm):
    f32 = jnp.float32
    wa[...] = jnp.dot(ws1[...], ws2[...], preferred_element_type=f32)
    wb[...] = jnp.dot(wn1[...], ws2[...], preferred_element_type=f32) + jnp.dot(
        ws1[...], wn2[...], preferred_element_type=f32
    )
    wc[...] = jnp.dot(wn1[...], wn2[...], preferred_element_type=f32)
    cm[0:1, :] = jnp.dot(b1[...], ws2[...], preferred_element_type=f32) + b2[...]
    cm[1:2, :] = jnp.dot(b1[...], wn2[...], preferred_element_type=f32)


def _wcomb(Ws1, Wn1, Ws2, Wn2, b1, b2):
    sh = jax.ShapeDtypeStruct
    return pl.pallas_call(
        _wcomb_body,
        out_shape=[
            sh((D_IN, NCLS), jnp.float32),
            sh((D_IN, NCLS), jnp.float32),
            sh((D_IN, NCLS), jnp.float32),
            sh((2, NCLS), jnp.float32),
        ],
    )(Ws1, Wn1, Ws2, Wn2, b1, b2)


_R2 = 2000


def _final_body(f_ref, m_ref, p2_ref, d_ref, wa_ref, wb_ref, wc_ref, cm_ref, o_ref):
    deg = d_ref[...]
    dmax = jnp.maximum(deg, 1.0)
    m1m = (p2_ref[0] + p2_ref[1]) / dmax
    r = (deg > 0.0).astype(jnp.float32)
    f32 = jnp.float32
    acc = jnp.dot(f_ref[...], wa_ref[...], preferred_element_type=f32)
    acc += jnp.dot(m_ref[...], wb_ref[...], preferred_element_type=f32)
    acc += jnp.dot(m1m, wc_ref[...], preferred_element_type=f32)
    acc += cm_ref[0:1, :] + r * cm_ref[1:2, :]
    o_ref[...] = acc


def _final(features, m1tab, p2, deg, wa, wb, wc, cm):
    return pl.pallas_call(
        _final_body,
        grid=(N_NODES // _R2,),
        in_specs=[
            pl.BlockSpec((_R2, D_IN), lambda i: (i, 0)),
            pl.BlockSpec((_R2, D_IN), lambda i: (i, 0)),
            pl.BlockSpec((NC, _R2, D_IN), lambda i: (0, i, 0)),
            pl.BlockSpec((_R2, 1), lambda i: (i, 0)),
            pl.BlockSpec((D_IN, NCLS), lambda i: (0, 0)),
            pl.BlockSpec((D_IN, NCLS), lambda i: (0, 0)),
            pl.BlockSpec((D_IN, NCLS), lambda i: (0, 0)),
            pl.BlockSpec((2, NCLS), lambda i: (0, 0)),
        ],
        out_specs=pl.BlockSpec((_R2, NCLS), lambda i: (i, 0)),
        out_shape=jax.ShapeDtypeStruct((N_NODES, NCLS), jnp.float32),
    )(features, m1tab, p2, deg, wa, wb, wc, cm)


def kernel(features, edge_index, W_self1, W_neigh1, b1, W_self2, W_neigh2, b2):
    src = edge_index[0].astype(jnp.int32)
    dst = edge_index[1].astype(jnp.int32)

    ones = jnp.ones((N_NODES, 1), jnp.float32)
    zpad = jnp.zeros((N_NODES, D_TAB - D_IN - 1), jnp.float32)
    ftab = jnp.concatenate([features, ones, zpad], axis=1)
    ftab = jnp.pad(ftab, ((0, N_PAD - N_NODES), (0, 0)))

    p1 = _agg_tab(ftab, src, dst)
    m1tab, deg = _combine(p1)
    p2 = _agg_feat(m1tab, src, dst)
    wa, wb, wc, cm = _wcomb(
        W_self1, W_neigh1, W_self2, W_neigh2,
        b1.reshape(1, -1), b2.reshape(1, -1),
    )
    return _final(features, m1tab, p2, deg, wa, wb, wc, cm)


# speedup vs baseline: 4.3119x; 1.3114x over previous
import functools

import jax
import jax.numpy as jnp
from jax import lax
from jax.experimental import pallas as pl
from jax.experimental.pallas import tpu as pltpu
from jax.experimental.pallas import tpu_sc as plsc

N_NODES = 10000
N_PAD = 10240
E = 320000
D_IN = 128
D_TAB = 144
D_HID = 256
NCLS = 47

NC = 2
NS = 16
NW = NC * NS
EPW = E // NW
B = 80
NCHUNK = EPW // B
ROWS_PT = N_PAD // NS


def _make_agg(D):
    mesh = plsc.VectorSubcoreMesh(core_axis_name="c", subcore_axis_name="s")

    DEPTH = 3
    IDEPTH = 4
    UNROLL = 12

    @functools.partial(
        pl.kernel,
        mesh=mesh,
        compiler_params=pltpu.CompilerParams(use_tc_tiling_on_sc=False),
        out_type=jax.ShapeDtypeStruct((NC, N_PAD, D), jnp.float32),
        scratch_types=(
            [pltpu.VMEM((B,), jnp.int32)] * IDEPTH
            + [pltpu.VMEM((B,), jnp.int32)] * IDEPTH
            + [pltpu.VMEM((B, D), jnp.float32)] * DEPTH
            + [pltpu.VMEM_SHARED((N_PAD, D), jnp.float32)]
            + [pltpu.SemaphoreType.DMA] * (IDEPTH + 2 * DEPTH)
        ),
    )
    def agg(table_hbm, src_hbm, dst_hbm, out_hbm, *refs):
        src_v = list(refs[0:IDEPTH])
        dst_v = list(refs[IDEPTH:2 * IDEPTH])
        bufs = list(refs[2 * IDEPTH:2 * IDEPTH + DEPTH])
        acc_sh = refs[2 * IDEPTH + DEPTH]
        sems = refs[2 * IDEPTH + DEPTH + 1:]
        isem = list(sems[0:IDEPTH])
        gsem = list(sems[IDEPTH:IDEPTH + DEPTH])
        ssem = list(sems[IDEPTH + DEPTH:])
        c = lax.axis_index("c")
        s = lax.axis_index("s")
        wid = s * NC + c
        buf0 = bufs[0]

        def zrow(i, carry):
            for b in range(D // 16):
                buf0[i, pl.ds(b * 16, 16)] = jnp.zeros((16,), jnp.float32)
            return carry

        lax.fori_loop(0, B, zrow, 0)
        for j in range(ROWS_PT // B):
            pltpu.sync_copy(buf0, acc_sh.at[pl.ds(s * ROWS_PT + j * B, B)])
        plsc.subcore_barrier()

        base0 = wid * EPW

        def idx_load(k, iu):
            pltpu.async_copy(src_hbm.at[pl.ds(base0 + k * B, B)], src_v[iu],
                             isem[iu])
            pltpu.async_copy(dst_hbm.at[pl.ds(base0 + k * B, B)], dst_v[iu],
                             isem[iu])

        def iwait(iu):
            pltpu.make_async_copy(
                src_hbm.at[pl.ds(0, B)], src_v[iu], isem[iu]).wait()
            pltpu.make_async_copy(
                dst_hbm.at[pl.ds(0, B)], dst_v[iu], isem[iu]).wait()

        def gather(u, iu):
            pltpu.async_copy(table_hbm.at[src_v[iu]], bufs[u], gsem[u])

        def gwait(u, iu):
            pltpu.make_async_copy(
                table_hbm.at[src_v[iu]], bufs[u], gsem[u]).wait()

        def scat_start(u, iu):
            pltpu.async_copy(bufs[u], acc_sh.at[dst_v[iu]], ssem[u], add=True)

        def swait(u, iu):
            pltpu.make_async_copy(
                bufs[u], acc_sh.at[dst_v[iu]], ssem[u]).wait()

        for k in range(DEPTH):
            idx_load(k, k)
        for k in range(DEPTH - 1):
            iwait(k)
            gather(k, k)

        def body(j, carry):
            for t in range(UNROLL):
                cur = UNROLL * j + t
                u = t % DEPTH
                iu = t % IDEPTH
                nxt = cur + DEPTH - 1
                pf = cur + IDEPTH - 1
                u_n = (t + DEPTH - 1) % DEPTH
                iu_n = (t + DEPTH - 1) % IDEPTH
                iu_p = (t + IDEPTH - 1) % IDEPTH

                @pl.when(cur < NCHUNK)
                def _():
                    gwait(u, iu)
                    scat_start(u, iu)

                @pl.when(nxt < NCHUNK)
                def _():
                    @pl.when(nxt >= DEPTH)
                    def _():
                        swait(u_n, iu_p)

                    @pl.when(pf < NCHUNK)
                    def _():
                        idx_load(pf, iu_p)

                    iwait(iu_n)
                    gather(u_n, iu_n)

            return carry

        nbody = (NCHUNK + UNROLL - 1) // UNROLL
        lax.fori_loop(0, nbody, body, 0)
        last = NCHUNK - 1
        for d in range(DEPTH):
            k = last - d
            swait(k % DEPTH, k % IDEPTH)

        plsc.subcore_barrier()
        pltpu.sync_copy(
            acc_sh.at[pl.ds(s * ROWS_PT, ROWS_PT)],
            out_hbm.at[c, pl.ds(s * ROWS_PT, ROWS_PT)],
        )

    return agg


_agg_tab = _make_agg(D_TAB)
_agg_feat = _make_agg(D_IN)


def _combine_body(p_ref, m_ref, d_ref):
    s = p_ref[0] + p_ref[1]
    deg = s[:, 128:129]
    m_ref[...] = s[:, :128] / jnp.maximum(deg, 1.0)
    d_ref[...] = deg


_R1 = 2048


def _combine(p1):
    return pl.pallas_call(
        _combine_body,
        grid=(N_PAD // _R1,),
        in_specs=[pl.BlockSpec((NC, _R1, D_TAB), lambda i: (0, i, 0))],
        out_specs=[
            pl.BlockSpec((_R1, D_IN), lambda i: (i, 0)),
            pl.BlockSpec((_R1, 1), lambda i: (i, 0)),
        ],
        out_shape=[
            jax.ShapeDtypeStruct((N_PAD, D_IN), jnp.float32),
            jax.ShapeDtypeStruct((N_PAD, 1), jnp.float32),
        ],
    )(p1)


_R2 = 2000


def _final_body(f_ref, m_ref, p2_ref, d_ref, ws1, wn1, ws2, wn2, b1, b2, o_ref):
    f32 = jnp.float32

    def mm(a, b):
        return jnp.dot(a, b, preferred_element_type=f32)

    wa = mm(ws1[...], ws2[...])
    wb = mm(wn1[...], ws2[...]) + mm(ws1[...], wn2[...])
    wc = mm(wn1[...], wn2[...])
    c0 = mm(b1[...], ws2[...]) + b2[...]
    c1 = mm(b1[...], wn2[...])

    deg = d_ref[...]
    dmax = jnp.maximum(deg, 1.0)
    m1m = (p2_ref[0] + p2_ref[1]) / dmax
    r = (deg > 0.0).astype(f32)
    acc = mm(f_ref[...], wa) + mm(m_ref[...], wb) + mm(m1m, wc)
    acc += c0 + r * c1
    o_ref[...] = acc


def _final(features, m1tab, p2, deg, Ws1, Wn1, Ws2, Wn2, b1, b2):
    return pl.pallas_call(
        _final_body,
        grid=(N_NODES // _R2,),
        in_specs=[
            pl.BlockSpec((_R2, D_IN), lambda i: (i, 0)),
            pl.BlockSpec((_R2, D_IN), lambda i: (i, 0)),
            pl.BlockSpec((NC, _R2, D_IN), lambda i: (0, i, 0)),
            pl.BlockSpec((_R2, 1), lambda i: (i, 0)),
            pl.BlockSpec((D_IN, D_HID), lambda i: (0, 0)),
            pl.BlockSpec((D_IN, D_HID), lambda i: (0, 0)),
            pl.BlockSpec((D_HID, NCLS), lambda i: (0, 0)),
            pl.BlockSpec((D_HID, NCLS), lambda i: (0, 0)),
            pl.BlockSpec((1, D_HID), lambda i: (0, 0)),
            pl.BlockSpec((1, NCLS), lambda i: (0, 0)),
        ],
        out_specs=pl.BlockSpec((_R2, NCLS), lambda i: (i, 0)),
        out_shape=jax.ShapeDtypeStruct((N_NODES, NCLS), jnp.float32),
    )(features, m1tab, p2, deg, Ws1, Wn1, Ws2, Wn2, b1, b2)


def kernel(features, edge_index, W_self1, W_neigh1, b1, W_self2, W_neigh2, b2):
    src = edge_index[0].astype(jnp.int32)
    dst = edge_index[1].astype(jnp.int32)

    ones = jnp.ones((N_NODES, 1), jnp.float32)
    zpad = jnp.zeros((N_NODES, D_TAB - D_IN - 1), jnp.float32)
    ftab = jnp.concatenate([features, ones, zpad], axis=1)
    ftab = jnp.pad(ftab, ((0, N_PAD - N_NODES), (0, 0)))

    p1 = _agg_tab(ftab, src, dst)
    m1tab, deg = _combine(p1)
    p2 = _agg_feat(m1tab, src, dst)
    return _final(features, m1tab, p2, deg,
                  W_self1, W_neigh1, W_self2, W_neigh2,
                  b1.reshape(1, -1), b2.reshape(1, -1))


# speedup vs baseline: 4.8984x; 1.1360x over previous
import functools

import jax
import jax.numpy as jnp
from jax import lax
from jax.experimental import pallas as pl
from jax.experimental.pallas import tpu as pltpu
from jax.experimental.pallas import tpu_sc as plsc

N_NODES = 10000
N_PAD = 10240
E = 320000
D_IN = 128
D_TAB = 144
D_HID = 256
NCLS = 47

NC = 2
NS = 16
NW = NC * NS
EPW = E // NW
B = 80
NCHUNK = EPW // B
ROWS_PT = N_PAD // NS


D_DEG = 16


def _make_agg(nrows, with_deg):
    mesh = plsc.VectorSubcoreMesh(core_axis_name="c", subcore_axis_name="s")

    DEPTH = 3
    IDEPTH = 4
    UNROLL = 12
    D = D_IN

    out_type = [jax.ShapeDtypeStruct((NC, N_PAD, D), jnp.float32)]
    scratch = (
        [pltpu.VMEM((B,), jnp.int32)] * IDEPTH
        + [pltpu.VMEM((B,), jnp.int32)] * IDEPTH
        + [pltpu.VMEM((B, D), jnp.float32)] * DEPTH
        + [pltpu.VMEM_SHARED((N_PAD, D), jnp.float32)]
        + [pltpu.SemaphoreType.DMA] * (IDEPTH + 2 * DEPTH)
    )
    if with_deg:
        out_type.append(jax.ShapeDtypeStruct((NC, N_PAD, D_DEG), jnp.float32))
        scratch += (
            [pltpu.VMEM((B, D_DEG), jnp.float32)]
            + [pltpu.VMEM_SHARED((N_PAD, D_DEG), jnp.float32)]
            + [pltpu.SemaphoreType.DMA] * DEPTH
        )

    @functools.partial(
        pl.kernel,
        mesh=mesh,
        compiler_params=pltpu.CompilerParams(use_tc_tiling_on_sc=False),
        out_type=out_type,
        scratch_types=scratch,
    )
    def agg(table_hbm, src_hbm, dst_hbm, *outs_refs):
        if with_deg:
            out_hbm, dout_hbm = outs_refs[0], outs_refs[1]
            refs = outs_refs[2:]
        else:
            out_hbm = outs_refs[0]
            refs = outs_refs[1:]
        src_v = list(refs[0:IDEPTH])
        dst_v = list(refs[IDEPTH:2 * IDEPTH])
        bufs = list(refs[2 * IDEPTH:2 * IDEPTH + DEPTH])
        acc_sh = refs[2 * IDEPTH + DEPTH]
        base = 2 * IDEPTH + DEPTH + 1
        isem = list(refs[base:base + IDEPTH])
        gsem = list(refs[base + IDEPTH:base + IDEPTH + DEPTH])
        ssem = list(refs[base + IDEPTH + DEPTH:base + IDEPTH + 2 * DEPTH])
        if with_deg:
            ones_v = refs[base + IDEPTH + 2 * DEPTH]
            dacc_sh = refs[base + IDEPTH + 2 * DEPTH + 1]
            dsem = list(refs[base + IDEPTH + 2 * DEPTH + 2:])
        c = lax.axis_index("c")
        s = lax.axis_index("s")
        wid = s * NC + c
        zbuf = bufs[DEPTH - 1]

        base0 = wid * EPW

        def idx_load(k, iu):
            pltpu.async_copy(src_hbm.at[pl.ds(base0 + k * B, B)], src_v[iu],
                             isem[iu])
            pltpu.async_copy(dst_hbm.at[pl.ds(base0 + k * B, B)], dst_v[iu],
                             isem[iu])

        def iwait(iu):
            pltpu.make_async_copy(
                src_hbm.at[pl.ds(0, B)], src_v[iu], isem[iu]).wait()
            pltpu.make_async_copy(
                dst_hbm.at[pl.ds(0, B)], dst_v[iu], isem[iu]).wait()

        def gather(u, iu):
            pltpu.async_copy(table_hbm.at[src_v[iu]], bufs[u], gsem[u])

        def gwait(u, iu):
            pltpu.make_async_copy(
                table_hbm.at[src_v[iu]], bufs[u], gsem[u]).wait()

        def scat_start(u, iu):
            pltpu.async_copy(bufs[u], acc_sh.at[dst_v[iu]], ssem[u], add=True)
            if with_deg:
                pltpu.async_copy(ones_v, dacc_sh.at[dst_v[iu]], dsem[u],
                                 add=True)

        def swait(u, iu):
            pltpu.make_async_copy(
                bufs[u], acc_sh.at[dst_v[iu]], ssem[u]).wait()
            if with_deg:
                pltpu.make_async_copy(
                    ones_v, dacc_sh.at[dst_v[iu]], dsem[u]).wait()

        for k in range(DEPTH):
            idx_load(k, k)
        for k in range(DEPTH - 1):
            iwait(k)
            gather(k, k)

        def zrow(i, carry):
            for b in range(D // 16):
                zbuf[i, pl.ds(b * 16, 16)] = jnp.zeros((16,), jnp.float32)
            return carry

        lax.fori_loop(0, B, zrow, 0)
        for j in range(ROWS_PT // B):
            pltpu.sync_copy(zbuf, acc_sh.at[pl.ds(s * ROWS_PT + j * B, B)])
        if with_deg:
            def zdrow(i, carry):
                ones_v[i, pl.ds(0, 16)] = jnp.zeros((16,), jnp.float32)
                return carry

            lax.fori_loop(0, B, zdrow, 0)
            for j in range(ROWS_PT // B):
                pltpu.sync_copy(
                    ones_v, dacc_sh.at[pl.ds(s * ROWS_PT + j * B, B)])

            def orow(i, carry):
                ones_v[i, pl.ds(0, 16)] = jnp.ones((16,), jnp.float32)
                return carry

            lax.fori_loop(0, B, orow, 0)
        plsc.subcore_barrier()

        def body(j, carry):
            for t in range(UNROLL):
                cur = UNROLL * j + t
                u = t % DEPTH
                iu = t % IDEPTH
                nxt = cur + DEPTH - 1
                pf = cur + IDEPTH - 1
                u_n = (t + DEPTH - 1) % DEPTH
                iu_n = (t + DEPTH - 1) % IDEPTH
                iu_p = (t + IDEPTH - 1) % IDEPTH

                @pl.when(cur < NCHUNK)
                def _():
                    gwait(u, iu)
                    scat_start(u, iu)

                @pl.when(nxt < NCHUNK)
                def _():
                    @pl.when(nxt >= DEPTH)
                    def _():
                        swait(u_n, iu_p)

                    @pl.when(pf < NCHUNK)
                    def _():
                        idx_load(pf, iu_p)

                    iwait(iu_n)
                    gather(u_n, iu_n)

            return carry

        nbody = (NCHUNK + UNROLL - 1) // UNROLL
        lax.fori_loop(0, nbody, body, 0)
        last = NCHUNK - 1
        for d in range(DEPTH):
            k = last - d
            swait(k % DEPTH, k % IDEPTH)

        plsc.subcore_barrier()
        pltpu.sync_copy(
            acc_sh.at[pl.ds(s * ROWS_PT, ROWS_PT)],
            out_hbm.at[c, pl.ds(s * ROWS_PT, ROWS_PT)],
        )
        if with_deg:
            pltpu.sync_copy(
                dacc_sh.at[pl.ds(s * ROWS_PT, ROWS_PT)],
                dout_hbm.at[c, pl.ds(s * ROWS_PT, ROWS_PT)],
            )

    return agg


_agg_deg = _make_agg(N_NODES, True)
_agg_plain = _make_agg(N_PAD, False)


def _combine_body(p_ref, pd_ref, m_ref, d_ref):
    s = p_ref[0] + p_ref[1]
    deg = pd_ref[0][:, 0:1] + pd_ref[1][:, 0:1]
    m_ref[...] = s / jnp.maximum(deg, 1.0)
    d_ref[...] = deg


_R1 = 2048


def _combine(p1, pd1):
    return pl.pallas_call(
        _combine_body,
        grid=(N_PAD // _R1,),
        in_specs=[
            pl.BlockSpec((NC, _R1, D_IN), lambda i: (0, i, 0)),
            pl.BlockSpec((NC, _R1, D_DEG), lambda i: (0, i, 0)),
        ],
        out_specs=[
            pl.BlockSpec((_R1, D_IN), lambda i: (i, 0)),
            pl.BlockSpec((_R1, 1), lambda i: (i, 0)),
        ],
        out_shape=[
            jax.ShapeDtypeStruct((N_PAD, D_IN), jnp.float32),
            jax.ShapeDtypeStruct((N_PAD, 1), jnp.float32),
        ],
    )(p1, pd1)


_R2 = 2000


def _final_body(f_ref, m_ref, p2_ref, d_ref, ws1, wn1, ws2, wn2, b1, b2, o_ref):
    f32 = jnp.float32

    def mm(a, b):
        return jnp.dot(a, b, preferred_element_type=f32)

    wa = mm(ws1[...], ws2[...])
    wb = mm(wn1[...], ws2[...]) + mm(ws1[...], wn2[...])
    wc = mm(wn1[...], wn2[...])
    c0 = mm(b1[...], ws2[...]) + b2[...]
    c1 = mm(b1[...], wn2[...])

    deg = d_ref[...]
    dmax = jnp.maximum(deg, 1.0)
    m1m = (p2_ref[0] + p2_ref[1]) / dmax
    r = (deg > 0.0).astype(f32)
    acc = mm(f_ref[...], wa) + mm(m_ref[...], wb) + mm(m1m, wc)
    acc += c0 + r * c1
    o_ref[...] = acc


def _final(features, m1tab, p2, deg, Ws1, Wn1, Ws2, Wn2, b1, b2):
    return pl.pallas_call(
        _final_body,
        grid=(N_NODES // _R2,),
        in_specs=[
            pl.BlockSpec((_R2, D_IN), lambda i: (i, 0)),
            pl.BlockSpec((_R2, D_IN), lambda i: (i, 0)),
            pl.BlockSpec((NC, _R2, D_IN), lambda i: (0, i, 0)),
            pl.BlockSpec((_R2, 1), lambda i: (i, 0)),
            pl.BlockSpec((D_IN, D_HID), lambda i: (0, 0)),
            pl.BlockSpec((D_IN, D_HID), lambda i: (0, 0)),
            pl.BlockSpec((D_HID, NCLS), lambda i: (0, 0)),
            pl.BlockSpec((D_HID, NCLS), lambda i: (0, 0)),
            pl.BlockSpec((1, D_HID), lambda i: (0, 0)),
            pl.BlockSpec((1, NCLS), lambda i: (0, 0)),
        ],
        out_specs=pl.BlockSpec((_R2, NCLS), lambda i: (i, 0)),
        out_shape=jax.ShapeDtypeStruct((N_NODES, NCLS), jnp.float32),
    )(features, m1tab, p2, deg, Ws1, Wn1, Ws2, Wn2, b1, b2)


def kernel(features, edge_index, W_self1, W_neigh1, b1, W_self2, W_neigh2, b2):
    src = edge_index[0].astype(jnp.int32)
    dst = edge_index[1].astype(jnp.int32)

    p1, pd1 = _agg_deg(features, src, dst)
    m1tab, deg = _combine(p1, pd1)
    p2 = _agg_plain(m1tab, src, dst)
    p2 = p2[0] if isinstance(p2, (list, tuple)) else p2
    return _final(features, m1tab, p2, deg,
                  W_self1, W_neigh1, W_self2, W_neigh2,
                  b1.reshape(1, -1), b2.reshape(1, -1))


# speedup vs baseline: 4.9889x; 1.0185x over previous
import functools

import jax
import jax.numpy as jnp
from jax import lax
from jax.experimental import pallas as pl
from jax.experimental.pallas import tpu as pltpu
from jax.experimental.pallas import tpu_sc as plsc

N_NODES = 10000
N_PAD = 10240
E = 320000
D_IN = 128
D_TAB = 144
D_HID = 256
NCLS = 47

NC = 2
NS = 16
NW = NC * NS
EPW = E // NW
B = 80
NCHUNK = EPW // B
ROWS_PT = N_PAD // NS


D_DEG = 16
D_YW = 96
D_Z = 48


def _make_agg(D, with_deg):
    mesh = plsc.VectorSubcoreMesh(core_axis_name="c", subcore_axis_name="s")

    DEPTH = 3
    IDEPTH = 4
    UNROLL = 12

    out_type = [jax.ShapeDtypeStruct((NC, N_PAD, D), jnp.float32)]
    scratch = (
        [pltpu.VMEM((B,), jnp.int32)] * IDEPTH
        + [pltpu.VMEM((B,), jnp.int32)] * IDEPTH
        + [pltpu.VMEM((B, D), jnp.float32)] * DEPTH
        + [pltpu.VMEM_SHARED((N_PAD, D), jnp.float32)]
        + [pltpu.SemaphoreType.DMA] * (IDEPTH + 2 * DEPTH)
    )
    if with_deg:
        out_type.append(jax.ShapeDtypeStruct((NC, N_PAD, D_DEG), jnp.float32))
        scratch += (
            [pltpu.VMEM((B, D_DEG), jnp.float32)]
            + [pltpu.VMEM_SHARED((N_PAD, D_DEG), jnp.float32)]
            + [pltpu.SemaphoreType.DMA] * DEPTH
        )

    @functools.partial(
        pl.kernel,
        mesh=mesh,
        compiler_params=pltpu.CompilerParams(use_tc_tiling_on_sc=False),
        out_type=out_type,
        scratch_types=scratch,
    )
    def agg(table_hbm, src_hbm, dst_hbm, *outs_refs):
        if with_deg:
            out_hbm, dout_hbm = outs_refs[0], outs_refs[1]
            refs = outs_refs[2:]
        else:
            out_hbm = outs_refs[0]
            refs = outs_refs[1:]
        src_v = list(refs[0:IDEPTH])
        dst_v = list(refs[IDEPTH:2 * IDEPTH])
        bufs = list(refs[2 * IDEPTH:2 * IDEPTH + DEPTH])
        acc_sh = refs[2 * IDEPTH + DEPTH]
        base = 2 * IDEPTH + DEPTH + 1
        isem = list(refs[base:base + IDEPTH])
        gsem = list(refs[base + IDEPTH:base + IDEPTH + DEPTH])
        ssem = list(refs[base + IDEPTH + DEPTH:base + IDEPTH + 2 * DEPTH])
        if with_deg:
            ones_v = refs[base + IDEPTH + 2 * DEPTH]
            dacc_sh = refs[base + IDEPTH + 2 * DEPTH + 1]
            dsem = list(refs[base + IDEPTH + 2 * DEPTH + 2:])
        c = lax.axis_index("c")
        s = lax.axis_index("s")
        wid = s * NC + c
        zbuf = bufs[DEPTH - 1]

        base0 = wid * EPW

        def idx_load(k, iu):
            pltpu.async_copy(src_hbm.at[pl.ds(base0 + k * B, B)], src_v[iu],
                             isem[iu])
            pltpu.async_copy(dst_hbm.at[pl.ds(base0 + k * B, B)], dst_v[iu],
                             isem[iu])

        def iwait(iu):
            pltpu.make_async_copy(
                src_hbm.at[pl.ds(0, B)], src_v[iu], isem[iu]).wait()
            pltpu.make_async_copy(
                dst_hbm.at[pl.ds(0, B)], dst_v[iu], isem[iu]).wait()

        def gather(u, iu):
            pltpu.async_copy(table_hbm.at[src_v[iu]], bufs[u], gsem[u])

        def gwait(u, iu):
            pltpu.make_async_copy(
                table_hbm.at[src_v[iu]], bufs[u], gsem[u]).wait()

        def scat_start(u, iu):
            pltpu.async_copy(bufs[u], acc_sh.at[dst_v[iu]], ssem[u], add=True)
            if with_deg:
                pltpu.async_copy(ones_v, dacc_sh.at[dst_v[iu]], dsem[u],
                                 add=True)

        def swait(u, iu):
            pltpu.make_async_copy(
                bufs[u], acc_sh.at[dst_v[iu]], ssem[u]).wait()
            if with_deg:
                pltpu.make_async_copy(
                    ones_v, dacc_sh.at[dst_v[iu]], dsem[u]).wait()

        for k in range(DEPTH):
            idx_load(k, k)
        for k in range(DEPTH - 1):
            iwait(k)
            gather(k, k)

        def zrow(i, carry):
            for b in range(D // 16):
                zbuf[i, pl.ds(b * 16, 16)] = jnp.zeros((16,), jnp.float32)
            return carry

        lax.fori_loop(0, B, zrow, 0)
        for j in range(ROWS_PT // B):
            pltpu.sync_copy(zbuf, acc_sh.at[pl.ds(s * ROWS_PT + j * B, B)])
        if with_deg:
            def zdrow(i, carry):
                ones_v[i, pl.ds(0, 16)] = jnp.zeros((16,), jnp.float32)
                return carry

            lax.fori_loop(0, B, zdrow, 0)
            for j in range(ROWS_PT // B):
                pltpu.sync_copy(
                    ones_v, dacc_sh.at[pl.ds(s * ROWS_PT + j * B, B)])

            def orow(i, carry):
                ones_v[i, pl.ds(0, 16)] = jnp.ones((16,), jnp.float32)
                return carry

            lax.fori_loop(0, B, orow, 0)
        plsc.subcore_barrier()

        def body(j, carry):
            for t in range(UNROLL):
                cur = UNROLL * j + t
                u = t % DEPTH
                iu = t % IDEPTH
                nxt = cur + DEPTH - 1
                pf = cur + IDEPTH - 1
                u_n = (t + DEPTH - 1) % DEPTH
                iu_n = (t + DEPTH - 1) % IDEPTH
                iu_p = (t + IDEPTH - 1) % IDEPTH

                @pl.when(cur < NCHUNK)
                def _():
                    gwait(u, iu)
                    scat_start(u, iu)

                @pl.when(nxt < NCHUNK)
                def _():
                    @pl.when(nxt >= DEPTH)
                    def _():
                        swait(u_n, iu_p)

                    @pl.when(pf < NCHUNK)
                    def _():
                        idx_load(pf, iu_p)

                    iwait(iu_n)
                    gather(u_n, iu_n)

            return carry

        nbody = (NCHUNK + UNROLL - 1) // UNROLL
        lax.fori_loop(0, nbody, body, 0)
        last = NCHUNK - 1
        for d in range(DEPTH):
            k = last - d
            swait(k % DEPTH, k % IDEPTH)

        plsc.subcore_barrier()
        pltpu.sync_copy(
            acc_sh.at[pl.ds(s * ROWS_PT, ROWS_PT)],
            out_hbm.at[c, pl.ds(s * ROWS_PT, ROWS_PT)],
        )
        if with_deg:
            pltpu.sync_copy(
                dacc_sh.at[pl.ds(s * ROWS_PT, ROWS_PT)],
                dout_hbm.at[c, pl.ds(s * ROWS_PT, ROWS_PT)],
            )

    return agg


_agg_deg = _make_agg(D_YW, True)
_agg_plain = _make_agg(D_Z, False)

_R0 = 2000


def _prepass_body(f_ref, ws1, wn1, ws2, wn2, b1, b2, yw_ref, u_ref, cm_ref):
    f32 = jnp.float32

    def mm(a, b):
        return jnp.dot(a, b, preferred_element_type=f32)

    wa = mm(ws1[...], ws2[...])
    wb = mm(wn1[...], ws2[...]) + mm(ws1[...], wn2[...])
    wc = mm(wn1[...], wn2[...])
    f = f_ref[...]
    z1 = jnp.zeros((_R0, 1), f32)
    yw_ref[...] = jnp.concatenate([mm(f, wb), z1, mm(f, wc), z1], axis=1)
    u_ref[...] = mm(f, wa)
    cm_ref[0:1, :] = mm(b1[...], ws2[...]) + b2[...]
    cm_ref[1:2, :] = mm(b1[...], wn2[...])


def _prepass(features, Ws1, Wn1, Ws2, Wn2, b1, b2):
    wspec = [
        pl.BlockSpec((D_IN, D_HID), lambda i: (0, 0)),
        pl.BlockSpec((D_IN, D_HID), lambda i: (0, 0)),
        pl.BlockSpec((D_HID, NCLS), lambda i: (0, 0)),
        pl.BlockSpec((D_HID, NCLS), lambda i: (0, 0)),
        pl.BlockSpec((1, D_HID), lambda i: (0, 0)),
        pl.BlockSpec((1, NCLS), lambda i: (0, 0)),
    ]
    return pl.pallas_call(
        _prepass_body,
        grid=(N_NODES // _R0,),
        in_specs=[pl.BlockSpec((_R0, D_IN), lambda i: (i, 0))] + wspec,
        out_specs=[
            pl.BlockSpec((_R0, D_YW), lambda i: (i, 0)),
            pl.BlockSpec((_R0, NCLS), lambda i: (i, 0)),
            pl.BlockSpec((2, NCLS), lambda i: (0, 0)),
        ],
        out_shape=[
            jax.ShapeDtypeStruct((N_NODES, D_YW), jnp.float32),
            jax.ShapeDtypeStruct((N_NODES, NCLS), jnp.float32),
            jax.ShapeDtypeStruct((2, NCLS), jnp.float32),
        ],
    )(features, Ws1, Wn1, Ws2, Wn2, b1, b2)


def _combine_body(p_ref, pd_ref, ym_ref, z_ref, d_ref):
    s = p_ref[0] + p_ref[1]
    deg = pd_ref[0][:, 0:1] + pd_ref[1][:, 0:1]
    dmax = jnp.maximum(deg, 1.0)
    ym_ref[...] = s[:, 0:D_Z] / dmax
    z_ref[...] = s[:, D_Z:D_YW] / dmax
    d_ref[...] = deg


_R1 = 2048


def _combine(p1, pd1):
    return pl.pallas_call(
        _combine_body,
        grid=(N_PAD // _R1,),
        in_specs=[
            pl.BlockSpec((NC, _R1, D_YW), lambda i: (0, i, 0)),
            pl.BlockSpec((NC, _R1, D_DEG), lambda i: (0, i, 0)),
        ],
        out_specs=[
            pl.BlockSpec((_R1, D_Z), lambda i: (i, 0)),
            pl.BlockSpec((_R1, D_Z), lambda i: (i, 0)),
            pl.BlockSpec((_R1, 1), lambda i: (i, 0)),
        ],
        out_shape=[
            jax.ShapeDtypeStruct((N_PAD, D_Z), jnp.float32),
            jax.ShapeDtypeStruct((N_PAD, D_Z), jnp.float32),
            jax.ShapeDtypeStruct((N_PAD, 1), jnp.float32),
        ],
    )(p1, pd1)


_R2 = 2000


def _final_body(u_ref, ym_ref, p2_ref, d_ref, cm_ref, o_ref):
    deg = d_ref[...]
    dmax = jnp.maximum(deg, 1.0)
    m2 = (p2_ref[0] + p2_ref[1])[:, 0:NCLS] / dmax
    r = (deg > 0.0).astype(jnp.float32)
    o_ref[...] = (u_ref[...] + ym_ref[:, 0:NCLS] + m2
                  + cm_ref[0:1, :] + r * cm_ref[1:2, :])


def _final(u, ym, p2, deg, cm):
    return pl.pallas_call(
        _final_body,
        grid=(N_NODES // _R2,),
        in_specs=[
            pl.BlockSpec((_R2, NCLS), lambda i: (i, 0)),
            pl.BlockSpec((_R2, D_Z), lambda i: (i, 0)),
            pl.BlockSpec((NC, _R2, D_Z), lambda i: (0, i, 0)),
            pl.BlockSpec((_R2, 1), lambda i: (i, 0)),
            pl.BlockSpec((2, NCLS), lambda i: (0, 0)),
        ],
        out_specs=pl.BlockSpec((_R2, NCLS), lambda i: (i, 0)),
        out_shape=jax.ShapeDtypeStruct((N_NODES, NCLS), jnp.float32),
    )(u, ym, p2, deg, cm)


def kernel(features, edge_index, W_self1, W_neigh1, b1, W_self2, W_neigh2, b2):
    src = edge_index[0].astype(jnp.int32)
    dst = edge_index[1].astype(jnp.int32)

    yw, u, cm = _prepass(features, W_self1, W_neigh1, W_self2, W_neigh2,
                         b1.reshape(1, -1), b2.reshape(1, -1))
    p1, pd1 = _agg_deg(yw, src, dst)
    ym, ztab, deg = _combine(p1, pd1)
    p2 = _agg_plain(ztab, src, dst)
    p2 = p2[0] if isinstance(p2, (list, tuple)) else p2
    return _final(u, ym, p2, deg, cm)


# speedup vs baseline: 5.0161x; 1.0054x over previous
import functools

import jax
import jax.numpy as jnp
from jax import lax
from jax.experimental import pallas as pl
from jax.experimental.pallas import tpu as pltpu
from jax.experimental.pallas import tpu_sc as plsc

N_NODES = 10000
N_PAD = 10240
E = 320000
D_IN = 128
D_TAB = 144
D_HID = 256
NCLS = 47

NC = 2
NS = 16
NW = NC * NS
EPW = E // NW
B = 80
NCHUNK = EPW // B
ROWS_PT = N_PAD // NS


D_YW = 112
D_Z = 48
C_DEG = 96


def _make_agg(D):
    mesh = plsc.VectorSubcoreMesh(core_axis_name="c", subcore_axis_name="s")

    DEPTH = 3
    IDEPTH = 4
    UNROLL = 12

    out_type = jax.ShapeDtypeStruct((NC, N_PAD, D), jnp.float32)
    scratch = (
        [pltpu.VMEM((B,), jnp.int32)] * IDEPTH
        + [pltpu.VMEM((B,), jnp.int32)] * IDEPTH
        + [pltpu.VMEM((B, D), jnp.float32)] * DEPTH
        + [pltpu.VMEM_SHARED((N_PAD, D), jnp.float32)]
        + [pltpu.SemaphoreType.DMA] * (IDEPTH + 2 * DEPTH)
    )

    @functools.partial(
        pl.kernel,
        mesh=mesh,
        compiler_params=pltpu.CompilerParams(use_tc_tiling_on_sc=False),
        out_type=out_type,
        scratch_types=scratch,
    )
    def agg(table_hbm, src_hbm, dst_hbm, out_hbm, *refs):
        src_v = list(refs[0:IDEPTH])
        dst_v = list(refs[IDEPTH:2 * IDEPTH])
        bufs = list(refs[2 * IDEPTH:2 * IDEPTH + DEPTH])
        acc_sh = refs[2 * IDEPTH + DEPTH]
        base = 2 * IDEPTH + DEPTH + 1
        isem = list(refs[base:base + IDEPTH])
        gsem = list(refs[base + IDEPTH:base + IDEPTH + DEPTH])
        ssem = list(refs[base + IDEPTH + DEPTH:base + IDEPTH + 2 * DEPTH])
        c = lax.axis_index("c")
        s = lax.axis_index("s")
        wid = s * NC + c
        zbuf = bufs[DEPTH - 1]

        base0 = wid * EPW

        def idx_load(k, iu):
            pltpu.async_copy(src_hbm.at[pl.ds(base0 + k * B, B)], src_v[iu],
                             isem[iu])
            pltpu.async_copy(dst_hbm.at[pl.ds(base0 + k * B, B)], dst_v[iu],
                             isem[iu])

        def iwait(iu):
            pltpu.make_async_copy(
                src_hbm.at[pl.ds(0, B)], src_v[iu], isem[iu]).wait()
            pltpu.make_async_copy(
                dst_hbm.at[pl.ds(0, B)], dst_v[iu], isem[iu]).wait()

        def gather(u, iu):
            pltpu.async_copy(table_hbm.at[src_v[iu]], bufs[u], gsem[u])

        def gwait(u, iu):
            pltpu.make_async_copy(
                table_hbm.at[src_v[iu]], bufs[u], gsem[u]).wait()

        def scat_start(u, iu):
            pltpu.async_copy(bufs[u], acc_sh.at[dst_v[iu]], ssem[u], add=True)

        def swait(u, iu):
            pltpu.make_async_copy(
                bufs[u], acc_sh.at[dst_v[iu]], ssem[u]).wait()

        for k in range(DEPTH):
            idx_load(k, k)
        for k in range(DEPTH - 1):
            iwait(k)
            gather(k, k)

        def zrow(i, carry):
            for b in range(D // 16):
                zbuf[i, pl.ds(b * 16, 16)] = jnp.zeros((16,), jnp.float32)
            return carry

        lax.fori_loop(0, B, zrow, 0)
        for j in range(ROWS_PT // B):
            pltpu.sync_copy(zbuf, acc_sh.at[pl.ds(s * ROWS_PT + j * B, B)])
        plsc.subcore_barrier()

        def body(j, carry):
            for t in range(UNROLL):
                cur = UNROLL * j + t
                u = t % DEPTH
                iu = t % IDEPTH
                nxt = cur + DEPTH - 1
                pf = cur + IDEPTH - 1
                u_n = (t + DEPTH - 1) % DEPTH
                iu_n = (t + DEPTH - 1) % IDEPTH
                iu_p = (t + IDEPTH - 1) % IDEPTH

                @pl.when(cur < NCHUNK)
                def _():
                    gwait(u, iu)
                    scat_start(u, iu)

                @pl.when(nxt < NCHUNK)
                def _():
                    @pl.when(nxt >= DEPTH)
                    def _():
                        swait(u_n, iu_p)

                    @pl.when(pf < NCHUNK)
                    def _():
                        idx_load(pf, iu_p)

                    iwait(iu_n)
                    gather(u_n, iu_n)

            return carry

        nbody = (NCHUNK + UNROLL - 1) // UNROLL
        lax.fori_loop(0, nbody, body, 0)
        last = NCHUNK - 1
        for d in range(DEPTH):
            k = last - d
            swait(k % DEPTH, k % IDEPTH)

        plsc.subcore_barrier()
        pltpu.sync_copy(
            acc_sh.at[pl.ds(s * ROWS_PT, ROWS_PT)],
            out_hbm.at[c, pl.ds(s * ROWS_PT, ROWS_PT)],
        )

    return agg


_agg_yw = _make_agg(D_YW)
_agg_z = _make_agg(D_Z)

_R0 = 2000


def _prepass_body(f_ref, ws1, wn1, ws2, wn2, b1, b2, yw_ref, u_ref, cm_ref):
    f32 = jnp.float32

    def mm(a, b):
        return jnp.dot(a, b, preferred_element_type=f32)

    wa = mm(ws1[...], ws2[...])
    wb = mm(wn1[...], ws2[...]) + mm(ws1[...], wn2[...])
    wc = mm(wn1[...], wn2[...])
    f = f_ref[...]
    z1 = jnp.zeros((_R0, 1), f32)
    one = jnp.ones((_R0, 1), f32)
    zp = jnp.zeros((_R0, D_YW - C_DEG - 1), f32)
    yw_ref[...] = jnp.concatenate(
        [mm(f, wb), z1, mm(f, wc), z1, one, zp], axis=1)
    u_ref[...] = mm(f, wa)
    cm_ref[0:1, :] = mm(b1[...], ws2[...]) + b2[...]
    cm_ref[1:2, :] = mm(b1[...], wn2[...])


def _prepass(features, Ws1, Wn1, Ws2, Wn2, b1, b2):
    wspec = [
        pl.BlockSpec((D_IN, D_HID), lambda i: (0, 0)),
        pl.BlockSpec((D_IN, D_HID), lambda i: (0, 0)),
        pl.BlockSpec((D_HID, NCLS), lambda i: (0, 0)),
        pl.BlockSpec((D_HID, NCLS), lambda i: (0, 0)),
        pl.BlockSpec((1, D_HID), lambda i: (0, 0)),
        pl.BlockSpec((1, NCLS), lambda i: (0, 0)),
    ]
    return pl.pallas_call(
        _prepass_body,
        grid=(N_NODES // _R0,),
        in_specs=[pl.BlockSpec((_R0, D_IN), lambda i: (i, 0))] + wspec,
        out_specs=[
            pl.BlockSpec((_R0, D_YW), lambda i: (i, 0)),
            pl.BlockSpec((_R0, NCLS), lambda i: (i, 0)),
            pl.BlockSpec((2, NCLS), lambda i: (0, 0)),
        ],
        out_shape=[
            jax.ShapeDtypeStruct((N_NODES, D_YW), jnp.float32),
            jax.ShapeDtypeStruct((N_NODES, NCLS), jnp.float32),
            jax.ShapeDtypeStruct((2, NCLS), jnp.float32),
        ],
    )(features, Ws1, Wn1, Ws2, Wn2, b1, b2)


def _combine_body(p_ref, ym_ref, z_ref, d_ref):
    s = p_ref[0] + p_ref[1]
    deg = s[:, C_DEG:C_DEG + 1]
    dmax = jnp.maximum(deg, 1.0)
    ym_ref[...] = s[:, 0:D_Z] / dmax
    z_ref[...] = s[:, D_Z:2 * D_Z] / dmax
    d_ref[...] = deg


_R1 = 2048


def _combine(p1):
    return pl.pallas_call(
        _combine_body,
        grid=(N_PAD // _R1,),
        in_specs=[
            pl.BlockSpec((NC, _R1, D_YW), lambda i: (0, i, 0)),
        ],
        out_specs=[
            pl.BlockSpec((_R1, D_Z), lambda i: (i, 0)),
            pl.BlockSpec((_R1, D_Z), lambda i: (i, 0)),
            pl.BlockSpec((_R1, 1), lambda i: (i, 0)),
        ],
        out_shape=[
            jax.ShapeDtypeStruct((N_PAD, D_Z), jnp.float32),
            jax.ShapeDtypeStruct((N_PAD, D_Z), jnp.float32),
            jax.ShapeDtypeStruct((N_PAD, 1), jnp.float32),
        ],
    )(p1)


_R2 = 2000


def _final_body(u_ref, ym_ref, p2_ref, d_ref, cm_ref, o_ref):
    deg = d_ref[...]
    dmax = jnp.maximum(deg, 1.0)
    m2 = (p2_ref[0] + p2_ref[1])[:, 0:NCLS] / dmax
    r = (deg > 0.0).astype(jnp.float32)
    o_ref[...] = (u_ref[...] + ym_ref[:, 0:NCLS] + m2
                  + cm_ref[0:1, :] + r * cm_ref[1:2, :])


def _final(u, ym, p2, deg, cm):
    return pl.pallas_call(
        _final_body,
        grid=(N_NODES // _R2,),
        in_specs=[
            pl.BlockSpec((_R2, NCLS), lambda i: (i, 0)),
            pl.BlockSpec((_R2, D_Z), lambda i: (i, 0)),
            pl.BlockSpec((NC, _R2, D_Z), lambda i: (0, i, 0)),
            pl.BlockSpec((_R2, 1), lambda i: (i, 0)),
            pl.BlockSpec((2, NCLS), lambda i: (0, 0)),
        ],
        out_specs=pl.BlockSpec((_R2, NCLS), lambda i: (i, 0)),
        out_shape=jax.ShapeDtypeStruct((N_NODES, NCLS), jnp.float32),
    )(u, ym, p2, deg, cm)


def kernel(features, edge_index, W_self1, W_neigh1, b1, W_self2, W_neigh2, b2):
    src = edge_index[0].astype(jnp.int32)
    dst = edge_index[1].astype(jnp.int32)

    yw, u, cm = _prepass(features, W_self1, W_neigh1, W_self2, W_neigh2,
                         b1.reshape(1, -1), b2.reshape(1, -1))
    p1 = _agg_yw(yw, src, dst)
    ym, ztab, deg = _combine(p1)
    p2 = _agg_z(ztab, src, dst)
    return _final(u, ym, p2, deg, cm)


# speedup vs baseline: 5.3144x; 1.0595x over previous
import functools

import jax
import jax.numpy as jnp
from jax import lax
from jax.experimental import pallas as pl
from jax.experimental.pallas import tpu as pltpu
from jax.experimental.pallas import tpu_sc as plsc

N_NODES = 10000
N_PAD = 10240
E = 320000
D_IN = 128
D_TAB = 144
D_HID = 256
NCLS = 47

NC = 2
NS = 16
NW = NC * NS
EPW = E // NW
B = 80
NCHUNK = EPW // B
ROWS_PT = N_PAD // NS


D_YW = 112
D_Z = 48
C_DEG = 96


def _make_agg(D):
    mesh = plsc.VectorSubcoreMesh(core_axis_name="c", subcore_axis_name="s")

    DEPTH = 4
    IDEPTH = 5
    UNROLL = 20

    out_type = jax.ShapeDtypeStruct((NC, N_PAD, D), jnp.float32)
    scratch = (
        [pltpu.VMEM((B,), jnp.int32)] * IDEPTH
        + [pltpu.VMEM((B,), jnp.int32)] * IDEPTH
        + [pltpu.VMEM((B, D), jnp.float32)] * DEPTH
        + [pltpu.VMEM_SHARED((N_PAD, D), jnp.float32)]
        + [pltpu.SemaphoreType.DMA] * (IDEPTH + 2 * DEPTH)
    )

    @functools.partial(
        pl.kernel,
        mesh=mesh,
        compiler_params=pltpu.CompilerParams(use_tc_tiling_on_sc=False),
        out_type=out_type,
        scratch_types=scratch,
    )
    def agg(table_hbm, src_hbm, dst_hbm, out_hbm, *refs):
        src_v = list(refs[0:IDEPTH])
        dst_v = list(refs[IDEPTH:2 * IDEPTH])
        bufs = list(refs[2 * IDEPTH:2 * IDEPTH + DEPTH])
        acc_sh = refs[2 * IDEPTH + DEPTH]
        base = 2 * IDEPTH + DEPTH + 1
        isem = list(refs[base:base + IDEPTH])
        gsem = list(refs[base + IDEPTH:base + IDEPTH + DEPTH])
        ssem = list(refs[base + IDEPTH + DEPTH:base + IDEPTH + 2 * DEPTH])
        c = lax.axis_index("c")
        s = lax.axis_index("s")
        wid = s * NC + c
        zbuf = bufs[DEPTH - 1]

        base0 = wid * EPW

        def idx_load(k, iu):
            pltpu.async_copy(src_hbm.at[pl.ds(base0 + k * B, B)], src_v[iu],
                             isem[iu])
            pltpu.async_copy(dst_hbm.at[pl.ds(base0 + k * B, B)], dst_v[iu],
                             isem[iu])

        def iwait(iu):
            pltpu.make_async_copy(
                src_hbm.at[pl.ds(0, B)], src_v[iu], isem[iu]).wait()
            pltpu.make_async_copy(
                dst_hbm.at[pl.ds(0, B)], dst_v[iu], isem[iu]).wait()

        def gather(u, iu):
            pltpu.async_copy(table_hbm.at[src_v[iu]], bufs[u], gsem[u])

        def gwait(u, iu):
            pltpu.make_async_copy(
                table_hbm.at[src_v[iu]], bufs[u], gsem[u]).wait()

        def scat_start(u, iu):
            pltpu.async_copy(bufs[u], acc_sh.at[dst_v[iu]], ssem[u], add=True)

        def swait(u, iu):
            pltpu.make_async_copy(
                bufs[u], acc_sh.at[dst_v[iu]], ssem[u]).wait()

        for k in range(DEPTH):
            idx_load(k, k)
        for k in range(DEPTH - 1):
            iwait(k)
            gather(k, k)

        def zrow(i, carry):
            for b in range(D // 16):
                zbuf[i, pl.ds(b * 16, 16)] = jnp.zeros((16,), jnp.float32)
            return carry

        lax.fori_loop(0, B, zrow, 0)
        for j in range(ROWS_PT // B):
            pltpu.sync_copy(zbuf, acc_sh.at[pl.ds(s * ROWS_PT + j * B, B)])
        plsc.subcore_barrier()

        def body(j, carry):
            for t in range(UNROLL):
                cur = UNROLL * j + t
                u = t % DEPTH
                iu = t % IDEPTH
                nxt = cur + DEPTH - 1
                pf = cur + IDEPTH - 1
                u_n = (t + DEPTH - 1) % DEPTH
                iu_n = (t + DEPTH - 1) % IDEPTH
                iu_p = (t + IDEPTH - 1) % IDEPTH

                @pl.when(cur < NCHUNK)
                def _():
                    gwait(u, iu)
                    scat_start(u, iu)

                @pl.when(nxt < NCHUNK)
                def _():
                    @pl.when(nxt >= DEPTH)
                    def _():
                        swait(u_n, iu_p)

                    @pl.when(pf < NCHUNK)
                    def _():
                        idx_load(pf, iu_p)

                    iwait(iu_n)
                    gather(u_n, iu_n)

            return carry

        nbody = (NCHUNK + UNROLL - 1) // UNROLL
        lax.fori_loop(0, nbody, body, 0)
        last = NCHUNK - 1
        for d in range(DEPTH):
            k = last - d
            swait(k % DEPTH, k % IDEPTH)

        plsc.subcore_barrier()
        pltpu.sync_copy(
            acc_sh.at[pl.ds(s * ROWS_PT, ROWS_PT)],
            out_hbm.at[c, pl.ds(s * ROWS_PT, ROWS_PT)],
        )

    return agg


_agg_yw = _make_agg(D_YW)
_agg_z = _make_agg(D_Z)

_R0 = 2000


def _prepass_body(f_ref, ws1, wn1, ws2, wn2, b1, b2, yw_ref, u_ref, cm_ref):
    f32 = jnp.float32

    def mm(a, b):
        return jnp.dot(a, b, preferred_element_type=f32)

    wa = mm(ws1[...], ws2[...])
    wb = mm(wn1[...], ws2[...]) + mm(ws1[...], wn2[...])
    wc = mm(wn1[...], wn2[...])
    f = f_ref[...]
    z1 = jnp.zeros((_R0, 1), f32)
    one = jnp.ones((_R0, 1), f32)
    zp = jnp.zeros((_R0, D_YW - C_DEG - 1), f32)
    yw_ref[...] = jnp.concatenate(
        [mm(f, wb), z1, mm(f, wc), z1, one, zp], axis=1)
    u_ref[...] = mm(f, wa)
    cm_ref[0:1, :] = mm(b1[...], ws2[...]) + b2[...]
    cm_ref[1:2, :] = mm(b1[...], wn2[...])


def _prepass(features, Ws1, Wn1, Ws2, Wn2, b1, b2):
    wspec = [
        pl.BlockSpec((D_IN, D_HID), lambda i: (0, 0)),
        pl.BlockSpec((D_IN, D_HID), lambda i: (0, 0)),
        pl.BlockSpec((D_HID, NCLS), lambda i: (0, 0)),
        pl.BlockSpec((D_HID, NCLS), lambda i: (0, 0)),
        pl.BlockSpec((1, D_HID), lambda i: (0, 0)),
        pl.BlockSpec((1, NCLS), lambda i: (0, 0)),
    ]
    return pl.pallas_call(
        _prepass_body,
        grid=(N_NODES // _R0,),
        in_specs=[pl.BlockSpec((_R0, D_IN), lambda i: (i, 0))] + wspec,
        out_specs=[
            pl.BlockSpec((_R0, D_YW), lambda i: (i, 0)),
            pl.BlockSpec((_R0, NCLS), lambda i: (i, 0)),
            pl.BlockSpec((2, NCLS), lambda i: (0, 0)),
        ],
        out_shape=[
            jax.ShapeDtypeStruct((N_NODES, D_YW), jnp.float32),
            jax.ShapeDtypeStruct((N_NODES, NCLS), jnp.float32),
            jax.ShapeDtypeStruct((2, NCLS), jnp.float32),
        ],
    )(features, Ws1, Wn1, Ws2, Wn2, b1, b2)


def _combine_body(p_ref, ym_ref, z_ref, d_ref):
    s = p_ref[0] + p_ref[1]
    deg = s[:, C_DEG:C_DEG + 1]
    dmax = jnp.maximum(deg, 1.0)
    ym_ref[...] = s[:, 0:D_Z] / dmax
    z_ref[...] = s[:, D_Z:2 * D_Z] / dmax
    d_ref[...] = deg


_R1 = 2048


def _combine(p1):
    return pl.pallas_call(
        _combine_body,
        grid=(N_PAD // _R1,),
        in_specs=[
            pl.BlockSpec((NC, _R1, D_YW), lambda i: (0, i, 0)),
        ],
        out_specs=[
            pl.BlockSpec((_R1, D_Z), lambda i: (i, 0)),
            pl.BlockSpec((_R1, D_Z), lambda i: (i, 0)),
            pl.BlockSpec((_R1, 1), lambda i: (i, 0)),
        ],
        out_shape=[
            jax.ShapeDtypeStruct((N_PAD, D_Z), jnp.float32),
            jax.ShapeDtypeStruct((N_PAD, D_Z), jnp.float32),
            jax.ShapeDtypeStruct((N_PAD, 1), jnp.float32),
        ],
    )(p1)


_R2 = 2000


def _final_body(u_ref, ym_ref, p2_ref, d_ref, cm_ref, o_ref):
    deg = d_ref[...]
    dmax = jnp.maximum(deg, 1.0)
    m2 = (p2_ref[0] + p2_ref[1])[:, 0:NCLS] / dmax
    r = (deg > 0.0).astype(jnp.float32)
    o_ref[...] = (u_ref[...] + ym_ref[:, 0:NCLS] + m2
                  + cm_ref[0:1, :] + r * cm_ref[1:2, :])


def _final(u, ym, p2, deg, cm):
    return pl.pallas_call(
        _final_body,
        grid=(N_NODES // _R2,),
        in_specs=[
            pl.BlockSpec((_R2, NCLS), lambda i: (i, 0)),
            pl.BlockSpec((_R2, D_Z), lambda i: (i, 0)),
            pl.BlockSpec((NC, _R2, D_Z), lambda i: (0, i, 0)),
            pl.BlockSpec((_R2, 1), lambda i: (i, 0)),
            pl.BlockSpec((2, NCLS), lambda i: (0, 0)),
        ],
        out_specs=pl.BlockSpec((_R2, NCLS), lambda i: (i, 0)),
        out_shape=jax.ShapeDtypeStruct((N_NODES, NCLS), jnp.float32),
    )(u, ym, p2, deg, cm)


def kernel(features, edge_index, W_self1, W_neigh1, b1, W_self2, W_neigh2, b2):
    src = edge_index[0].astype(jnp.int32)
    dst = edge_index[1].astype(jnp.int32)

    yw, u, cm = _prepass(features, W_self1, W_neigh1, W_self2, W_neigh2,
                         b1.reshape(1, -1), b2.reshape(1, -1))
    p1 = _agg_yw(yw, src, dst)
    ym, ztab, deg = _combine(p1)
    p2 = _agg_z(ztab, src, dst)
    return _final(u, ym, p2, deg, cm)


# speedup vs baseline: 5.5114x; 1.0371x over previous
import functools

import jax
import jax.numpy as jnp
from jax import lax
from jax.experimental import pallas as pl
from jax.experimental.pallas import tpu as pltpu
from jax.experimental.pallas import tpu_sc as plsc

N_NODES = 10000
N_PAD = 10240
E = 320000
D_IN = 128
D_TAB = 144
D_HID = 256
NCLS = 47

NC = 2
NS = 16
NW = NC * NS
EPW = E // NW
B = 80
NCHUNK = EPW // B
ROWS_PT = N_PAD // NS


D_YW = 112
D_Z = 48
C_DEG = 96


def _make_agg(D):
    mesh = plsc.VectorSubcoreMesh(core_axis_name="c", subcore_axis_name="s")

    DEPTH = 5
    IDEPTH = 6
    UNROLL = 30

    out_type = jax.ShapeDtypeStruct((NC, N_PAD, D), jnp.float32)
    scratch = (
        [pltpu.VMEM((B,), jnp.int32)] * IDEPTH
        + [pltpu.VMEM((B,), jnp.int32)] * IDEPTH
        + [pltpu.VMEM((B, D), jnp.float32)] * DEPTH
        + [pltpu.VMEM_SHARED((N_PAD, D), jnp.float32)]
        + [pltpu.SemaphoreType.DMA] * (IDEPTH + 2 * DEPTH)
    )

    @functools.partial(
        pl.kernel,
        mesh=mesh,
        compiler_params=pltpu.CompilerParams(use_tc_tiling_on_sc=False),
        out_type=out_type,
        scratch_types=scratch,
    )
    def agg(table_hbm, src_hbm, dst_hbm, out_hbm, *refs):
        src_v = list(refs[0:IDEPTH])
        dst_v = list(refs[IDEPTH:2 * IDEPTH])
        bufs = list(refs[2 * IDEPTH:2 * IDEPTH + DEPTH])
        acc_sh = refs[2 * IDEPTH + DEPTH]
        base = 2 * IDEPTH + DEPTH + 1
        isem = list(refs[base:base + IDEPTH])
        gsem = list(refs[base + IDEPTH:base + IDEPTH + DEPTH])
        ssem = list(refs[base + IDEPTH + DEPTH:base + IDEPTH + 2 * DEPTH])
        c = lax.axis_index("c")
        s = lax.axis_index("s")
        wid = s * NC + c
        zbuf = bufs[DEPTH - 1]

        base0 = wid * EPW

        def idx_load(k, iu):
            pltpu.async_copy(src_hbm.at[pl.ds(base0 + k * B, B)], src_v[iu],
                             isem[iu])
            pltpu.async_copy(dst_hbm.at[pl.ds(base0 + k * B, B)], dst_v[iu],
                             isem[iu])

        def iwait(iu):
            pltpu.make_async_copy(
                src_hbm.at[pl.ds(0, B)], src_v[iu], isem[iu]).wait()
            pltpu.make_async_copy(
                dst_hbm.at[pl.ds(0, B)], dst_v[iu], isem[iu]).wait()

        def gather(u, iu):
            pltpu.async_copy(table_hbm.at[src_v[iu]], bufs[u], gsem[u])

        def gwait(u, iu):
            pltpu.make_async_copy(
                table_hbm.at[src_v[iu]], bufs[u], gsem[u]).wait()

        def scat_start(u, iu):
            pltpu.async_copy(bufs[u], acc_sh.at[dst_v[iu]], ssem[u], add=True)

        def swait(u, iu):
            pltpu.make_async_copy(
                bufs[u], acc_sh.at[dst_v[iu]], ssem[u]).wait()

        for k in range(DEPTH):
            idx_load(k, k)
        for k in range(DEPTH - 1):
            iwait(k)
            gather(k, k)

        def zrow(i, carry):
            for b in range(D // 16):
                zbuf[i, pl.ds(b * 16, 16)] = jnp.zeros((16,), jnp.float32)
            return carry

        lax.fori_loop(0, B, zrow, 0)
        for j in range(ROWS_PT // B):
            pltpu.sync_copy(zbuf, acc_sh.at[pl.ds(s * ROWS_PT + j * B, B)])
        plsc.subcore_barrier()

        def body(j, carry):
            for t in range(UNROLL):
                cur = UNROLL * j + t
                u = t % DEPTH
                iu = t % IDEPTH
                nxt = cur + DEPTH - 1
                pf = cur + IDEPTH - 1
                u_n = (t + DEPTH - 1) % DEPTH
                iu_n = (t + DEPTH - 1) % IDEPTH
                iu_p = (t + IDEPTH - 1) % IDEPTH

                @pl.when(cur < NCHUNK)
                def _():
                    gwait(u, iu)
                    scat_start(u, iu)

                @pl.when(nxt < NCHUNK)
                def _():
                    @pl.when(nxt >= DEPTH)
                    def _():
                        swait(u_n, iu_p)

                    @pl.when(pf < NCHUNK)
                    def _():
                        idx_load(pf, iu_p)

                    iwait(iu_n)
                    gather(u_n, iu_n)

            return carry

        nbody = (NCHUNK + UNROLL - 1) // UNROLL
        lax.fori_loop(0, nbody, body, 0)
        last = NCHUNK - 1
        for d in range(DEPTH):
            k = last - d
            swait(k % DEPTH, k % IDEPTH)

        plsc.subcore_barrier()
        pltpu.sync_copy(
            acc_sh.at[pl.ds(s * ROWS_PT, ROWS_PT)],
            out_hbm.at[c, pl.ds(s * ROWS_PT, ROWS_PT)],
        )

    return agg


_agg_yw = _make_agg(D_YW)
_agg_z = _make_agg(D_Z)

_R0 = 2000


def _prepass_body(f_ref, ws1, wn1, ws2, wn2, b1, b2, yw_ref, u_ref, cm_ref):
    f32 = jnp.float32

    def mm(a, b):
        return jnp.dot(a, b, preferred_element_type=f32)

    wa = mm(ws1[...], ws2[...])
    wb = mm(wn1[...], ws2[...]) + mm(ws1[...], wn2[...])
    wc = mm(wn1[...], wn2[...])
    f = f_ref[...]
    z1 = jnp.zeros((_R0, 1), f32)
    one = jnp.ones((_R0, 1), f32)
    zp = jnp.zeros((_R0, D_YW - C_DEG - 1), f32)
    yw_ref[...] = jnp.concatenate(
        [mm(f, wb), z1, mm(f, wc), z1, one, zp], axis=1)
    u_ref[...] = mm(f, wa)
    cm_ref[0:1, :] = mm(b1[...], ws2[...]) + b2[...]
    cm_ref[1:2, :] = mm(b1[...], wn2[...])


def _prepass(features, Ws1, Wn1, Ws2, Wn2, b1, b2):
    wspec = [
        pl.BlockSpec((D_IN, D_HID), lambda i: (0, 0)),
        pl.BlockSpec((D_IN, D_HID), lambda i: (0, 0)),
        pl.BlockSpec((D_HID, NCLS), lambda i: (0, 0)),
        pl.BlockSpec((D_HID, NCLS), lambda i: (0, 0)),
        pl.BlockSpec((1, D_HID), lambda i: (0, 0)),
        pl.BlockSpec((1, NCLS), lambda i: (0, 0)),
    ]
    return pl.pallas_call(
        _prepass_body,
        grid=(N_NODES // _R0,),
        in_specs=[pl.BlockSpec((_R0, D_IN), lambda i: (i, 0))] + wspec,
        out_specs=[
            pl.BlockSpec((_R0, D_YW), lambda i: (i, 0)),
            pl.BlockSpec((_R0, NCLS), lambda i: (i, 0)),
            pl.BlockSpec((2, NCLS), lambda i: (0, 0)),
        ],
        out_shape=[
            jax.ShapeDtypeStruct((N_NODES, D_YW), jnp.float32),
            jax.ShapeDtypeStruct((N_NODES, NCLS), jnp.float32),
            jax.ShapeDtypeStruct((2, NCLS), jnp.float32),
        ],
    )(features, Ws1, Wn1, Ws2, Wn2, b1, b2)


def _combine_body(p_ref, ym_ref, z_ref, d_ref):
    s = p_ref[0] + p_ref[1]
    deg = s[:, C_DEG:C_DEG + 1]
    dmax = jnp.maximum(deg, 1.0)
    ym_ref[...] = s[:, 0:D_Z] / dmax
    z_ref[...] = s[:, D_Z:2 * D_Z] / dmax
    d_ref[...] = deg


_R1 = 2048


def _combine(p1):
    return pl.pallas_call(
        _combine_body,
        grid=(N_PAD // _R1,),
        in_specs=[
            pl.BlockSpec((NC, _R1, D_YW), lambda i: (0, i, 0)),
        ],
        out_specs=[
            pl.BlockSpec((_R1, D_Z), lambda i: (i, 0)),
            pl.BlockSpec((_R1, D_Z), lambda i: (i, 0)),
            pl.BlockSpec((_R1, 1), lambda i: (i, 0)),
        ],
        out_shape=[
            jax.ShapeDtypeStruct((N_PAD, D_Z), jnp.float32),
            jax.ShapeDtypeStruct((N_PAD, D_Z), jnp.float32),
            jax.ShapeDtypeStruct((N_PAD, 1), jnp.float32),
        ],
    )(p1)


_R2 = 2000


def _final_body(u_ref, ym_ref, p2_ref, d_ref, cm_ref, o_ref):
    deg = d_ref[...]
    dmax = jnp.maximum(deg, 1.0)
    m2 = (p2_ref[0] + p2_ref[1])[:, 0:NCLS] / dmax
    r = (deg > 0.0).astype(jnp.float32)
    o_ref[...] = (u_ref[...] + ym_ref[:, 0:NCLS] + m2
                  + cm_ref[0:1, :] + r * cm_ref[1:2, :])


def _final(u, ym, p2, deg, cm):
    return pl.pallas_call(
        _final_body,
        grid=(N_NODES // _R2,),
        in_specs=[
            pl.BlockSpec((_R2, NCLS), lambda i: (i, 0)),
            pl.BlockSpec((_R2, D_Z), lambda i: (i, 0)),
            pl.BlockSpec((NC, _R2, D_Z), lambda i: (0, i, 0)),
            pl.BlockSpec((_R2, 1), lambda i: (i, 0)),
            pl.BlockSpec((2, NCLS), lambda i: (0, 0)),
        ],
        out_specs=pl.BlockSpec((_R2, NCLS), lambda i: (i, 0)),
        out_shape=jax.ShapeDtypeStruct((N_NODES, NCLS), jnp.float32),
    )(u, ym, p2, deg, cm)


def kernel(features, edge_index, W_self1, W_neigh1, b1, W_self2, W_neigh2, b2):
    src = edge_index[0].astype(jnp.int32)
    dst = edge_index[1].astype(jnp.int32)

    yw, u, cm = _prepass(features, W_self1, W_neigh1, W_self2, W_neigh2,
                         b1.reshape(1, -1), b2.reshape(1, -1))
    p1 = _agg_yw(yw, src, dst)
    ym, ztab, deg = _combine(p1)
    p2 = _agg_z(ztab, src, dst)
    return _final(u, ym, p2, deg, cm)


# speedup vs baseline: 5.5554x; 1.0080x over previous
import functools

import jax
import jax.numpy as jnp
from jax import lax
from jax.experimental import pallas as pl
from jax.experimental.pallas import tpu as pltpu
from jax.experimental.pallas import tpu_sc as plsc

N_NODES = 10000
N_PAD = 10240
E = 320000
D_IN = 128
D_TAB = 144
D_HID = 256
NCLS = 47

NC = 2
NS = 16
NW = NC * NS
EPW = E // NW
B = 80
NCHUNK = EPW // B
ROWS_PT = N_PAD // NS


D_YW = 112
D_Z = 48
C_DEG = 96


def _make_agg(D):
    mesh = plsc.VectorSubcoreMesh(core_axis_name="c", subcore_axis_name="s")

    DEPTH = 6
    IDEPTH = 7
    UNROLL = 42

    out_type = jax.ShapeDtypeStruct((NC, N_PAD, D), jnp.float32)
    scratch = (
        [pltpu.VMEM((B,), jnp.int32)] * IDEPTH
        + [pltpu.VMEM((B,), jnp.int32)] * IDEPTH
        + [pltpu.VMEM((B, D), jnp.float32)] * DEPTH
        + [pltpu.VMEM_SHARED((N_PAD, D), jnp.float32)]
        + [pltpu.SemaphoreType.DMA] * (IDEPTH + 2 * DEPTH)
    )

    @functools.partial(
        pl.kernel,
        mesh=mesh,
        compiler_params=pltpu.CompilerParams(use_tc_tiling_on_sc=False),
        out_type=out_type,
        scratch_types=scratch,
    )
    def agg(table_hbm, src_hbm, dst_hbm, out_hbm, *refs):
        src_v = list(refs[0:IDEPTH])
        dst_v = list(refs[IDEPTH:2 * IDEPTH])
        bufs = list(refs[2 * IDEPTH:2 * IDEPTH + DEPTH])
        acc_sh = refs[2 * IDEPTH + DEPTH]
        base = 2 * IDEPTH + DEPTH + 1
        isem = list(refs[base:base + IDEPTH])
        gsem = list(refs[base + IDEPTH:base + IDEPTH + DEPTH])
        ssem = list(refs[base + IDEPTH + DEPTH:base + IDEPTH + 2 * DEPTH])
        c = lax.axis_index("c")
        s = lax.axis_index("s")
        wid = s * NC + c
        zbuf = bufs[DEPTH - 1]

        base0 = wid * EPW

        def idx_load(k, iu):
            pltpu.async_copy(src_hbm.at[pl.ds(base0 + k * B, B)], src_v[iu],
                             isem[iu])
            pltpu.async_copy(dst_hbm.at[pl.ds(base0 + k * B, B)], dst_v[iu],
                             isem[iu])

        def iwait(iu):
            pltpu.make_async_copy(
                src_hbm.at[pl.ds(0, B)], src_v[iu], isem[iu]).wait()
            pltpu.make_async_copy(
                dst_hbm.at[pl.ds(0, B)], dst_v[iu], isem[iu]).wait()

        def gather(u, iu):
            pltpu.async_copy(table_hbm.at[src_v[iu]], bufs[u], gsem[u])

        def gwait(u, iu):
            pltpu.make_async_copy(
                table_hbm.at[src_v[iu]], bufs[u], gsem[u]).wait()

        def scat_start(u, iu):
            pltpu.async_copy(bufs[u], acc_sh.at[dst_v[iu]], ssem[u], add=True)

        def swait(u, iu):
            pltpu.make_async_copy(
                bufs[u], acc_sh.at[dst_v[iu]], ssem[u]).wait()

        for k in range(DEPTH):
            idx_load(k, k)
        for k in range(DEPTH - 1):
            iwait(k)
            gather(k, k)

        def zrow(i, carry):
            for b in range(D // 16):
                zbuf[i, pl.ds(b * 16, 16)] = jnp.zeros((16,), jnp.float32)
            return carry

        lax.fori_loop(0, B, zrow, 0)
        for j in range(ROWS_PT // B):
            pltpu.sync_copy(zbuf, acc_sh.at[pl.ds(s * ROWS_PT + j * B, B)])
        plsc.subcore_barrier()

        def body(j, carry):
            for t in range(UNROLL):
                cur = UNROLL * j + t
                u = t % DEPTH
                iu = t % IDEPTH
                nxt = cur + DEPTH - 1
                pf = cur + IDEPTH - 1
                u_n = (t + DEPTH - 1) % DEPTH
                iu_n = (t + DEPTH - 1) % IDEPTH
                iu_p = (t + IDEPTH - 1) % IDEPTH

                @pl.when(cur < NCHUNK)
                def _():
                    gwait(u, iu)
                    scat_start(u, iu)

                @pl.when(nxt < NCHUNK)
                def _():
                    @pl.when(nxt >= DEPTH)
                    def _():
                        swait(u_n, iu_p)

                    @pl.when(pf < NCHUNK)
                    def _():
                        idx_load(pf, iu_p)

                    iwait(iu_n)
                    gather(u_n, iu_n)

            return carry

        nbody = (NCHUNK + UNROLL - 1) // UNROLL
        lax.fori_loop(0, nbody, body, 0)
        last = NCHUNK - 1
        for d in range(DEPTH):
            k = last - d
            swait(k % DEPTH, k % IDEPTH)

        plsc.subcore_barrier()
        pltpu.sync_copy(
            acc_sh.at[pl.ds(s * ROWS_PT, ROWS_PT)],
            out_hbm.at[c, pl.ds(s * ROWS_PT, ROWS_PT)],
        )

    return agg


_agg_yw = _make_agg(D_YW)
_agg_z = _make_agg(D_Z)

_R0 = 2000


def _prepass_body(f_ref, ws1, wn1, ws2, wn2, b1, b2, yw_ref, u_ref, cm_ref):
    f32 = jnp.float32

    def mm(a, b):
        return jnp.dot(a, b, preferred_element_type=f32)

    wa = mm(ws1[...], ws2[...])
    wb = mm(wn1[...], ws2[...]) + mm(ws1[...], wn2[...])
    wc = mm(wn1[...], wn2[...])
    f = f_ref[...]
    z1 = jnp.zeros((_R0, 1), f32)
    one = jnp.ones((_R0, 1), f32)
    zp = jnp.zeros((_R0, D_YW - C_DEG - 1), f32)
    yw_ref[...] = jnp.concatenate(
        [mm(f, wb), z1, mm(f, wc), z1, one, zp], axis=1)
    u_ref[...] = mm(f, wa)
    cm_ref[0:1, :] = mm(b1[...], ws2[...]) + b2[...]
    cm_ref[1:2, :] = mm(b1[...], wn2[...])


def _prepass(features, Ws1, Wn1, Ws2, Wn2, b1, b2):
    wspec = [
        pl.BlockSpec((D_IN, D_HID), lambda i: (0, 0)),
        pl.BlockSpec((D_IN, D_HID), lambda i: (0, 0)),
        pl.BlockSpec((D_HID, NCLS), lambda i: (0, 0)),
        pl.BlockSpec((D_HID, NCLS), lambda i: (0, 0)),
        pl.BlockSpec((1, D_HID), lambda i: (0, 0)),
        pl.BlockSpec((1, NCLS), lambda i: (0, 0)),
    ]
    return pl.pallas_call(
        _prepass_body,
        grid=(N_NODES // _R0,),
        in_specs=[pl.BlockSpec((_R0, D_IN), lambda i: (i, 0))] + wspec,
        out_specs=[
            pl.BlockSpec((_R0, D_YW), lambda i: (i, 0)),
            pl.BlockSpec((_R0, NCLS), lambda i: (i, 0)),
            pl.BlockSpec((2, NCLS), lambda i: (0, 0)),
        ],
        out_shape=[
            jax.ShapeDtypeStruct((N_NODES, D_YW), jnp.float32),
            jax.ShapeDtypeStruct((N_NODES, NCLS), jnp.float32),
            jax.ShapeDtypeStruct((2, NCLS), jnp.float32),
        ],
    )(features, Ws1, Wn1, Ws2, Wn2, b1, b2)


def _combine_body(p_ref, ym_ref, z_ref, d_ref):
    s = p_ref[0] + p_ref[1]
    deg = s[:, C_DEG:C_DEG + 1]
    dmax = jnp.maximum(deg, 1.0)
    ym_ref[...] = s[:, 0:D_Z] / dmax
    z_ref[...] = s[:, D_Z:2 * D_Z] / dmax
    d_ref[...] = deg


_R1 = 2048


def _combine(p1):
    return pl.pallas_call(
        _combine_body,
        grid=(N_PAD // _R1,),
        in_specs=[
            pl.BlockSpec((NC, _R1, D_YW), lambda i: (0, i, 0)),
        ],
        out_specs=[
            pl.BlockSpec((_R1, D_Z), lambda i: (i, 0)),
            pl.BlockSpec((_R1, D_Z), lambda i: (i, 0)),
            pl.BlockSpec((_R1, 1), lambda i: (i, 0)),
        ],
        out_shape=[
            jax.ShapeDtypeStruct((N_PAD, D_Z), jnp.float32),
            jax.ShapeDtypeStruct((N_PAD, D_Z), jnp.float32),
            jax.ShapeDtypeStruct((N_PAD, 1), jnp.float32),
        ],
    )(p1)


_R2 = 2000


def _final_body(u_ref, ym_ref, p2_ref, d_ref, cm_ref, o_ref):
    deg = d_ref[...]
    dmax = jnp.maximum(deg, 1.0)
    m2 = (p2_ref[0] + p2_ref[1])[:, 0:NCLS] / dmax
    r = (deg > 0.0).astype(jnp.float32)
    o_ref[...] = (u_ref[...] + ym_ref[:, 0:NCLS] + m2
                  + cm_ref[0:1, :] + r * cm_ref[1:2, :])


def _final(u, ym, p2, deg, cm):
    return pl.pallas_call(
        _final_body,
        grid=(N_NODES // _R2,),
        in_specs=[
            pl.BlockSpec((_R2, NCLS), lambda i: (i, 0)),
            pl.BlockSpec((_R2, D_Z), lambda i: (i, 0)),
            pl.BlockSpec((NC, _R2, D_Z), lambda i: (0, i, 0)),
            pl.BlockSpec((_R2, 1), lambda i: (i, 0)),
            pl.BlockSpec((2, NCLS), lambda i: (0, 0)),
        ],
        out_specs=pl.BlockSpec((_R2, NCLS), lambda i: (i, 0)),
        out_shape=jax.ShapeDtypeStruct((N_NODES, NCLS), jnp.float32),
    )(u, ym, p2, deg, cm)


def kernel(features, edge_index, W_self1, W_neigh1, b1, W_self2, W_neigh2, b2):
    src = edge_index[0].astype(jnp.int32)
    dst = edge_index[1].astype(jnp.int32)

    yw, u, cm = _prepass(features, W_self1, W_neigh1, W_self2, W_neigh2,
                         b1.reshape(1, -1), b2.reshape(1, -1))
    p1 = _agg_yw(yw, src, dst)
    ym, ztab, deg = _combine(p1)
    p2 = _agg_z(ztab, src, dst)
    return _final(u, ym, p2, deg, cm)
